# Initial kernel scaffold; baseline (speedup 1.0000x reference)
import jax
import jax.numpy as jnp
from jax.experimental import pallas as pl


def kernel(x, edge_index, clue_idx, W_g, b_g, alpha_left, alpha_right, W_ih, W_hh, b_ih, b_hh, W_cause, b_cause, W_effect, b_effect):
    raise NotImplementedError("write your pallas kernel here")


# speedup vs baseline: 9.8608x; 9.8608x over previous
import dataclasses
import functools

import jax
import jax.numpy as jnp
from jax import lax
from jax.experimental import pallas as pl
from jax.experimental.pallas import tpu as pltpu
from jax.experimental.pallas import tpu_sc as plsc

N = 10000
E = 160000
D = 256
H = 128
T = 64

NC = 2
NS = 16
NW = NC * NS
B = 128
NB = E // B
MAXT = (NB + NW - 1) // NW

NPAD = 10240
AGG_PAD = 81920
WMAT_PAD = 655360

_mesh = plsc.VectorSubcoreMesh(core_axis_name="c", subcore_axis_name="s")

_sc_params = pltpu.CompilerParams()
if "needs_layout_passes" in pltpu.CompilerParams.__dataclass_fields__:
  _sc_params = dataclasses.replace(_sc_params, needs_layout_passes=False)


def _zero_vmem(ref, n):
  z = jnp.zeros((16,), ref.dtype)

  @pl.loop(0, n, step=16)
  def _(i):
    ref[pl.ds(i, 16)] = z


@functools.partial(
    pl.kernel,
    out_type=(
        jax.ShapeDtypeStruct((E,), jnp.float32),
        jax.ShapeDtypeStruct((NC, NPAD), jnp.float32),
    ),
    mesh=_mesh,
    scratch_types=[
        pltpu.VMEM((N,), jnp.float32),
        pltpu.VMEM((N,), jnp.float32),
        pltpu.VMEM((B,), jnp.int32),
        pltpu.VMEM((B,), jnp.int32),
        pltpu.VMEM((B,), jnp.float32),
        pltpu.VMEM((640,), jnp.float32),
        pltpu.VMEM_SHARED((NPAD,), jnp.float32),
    ],
    compiler_params=_sc_params,
)
def _sc_pass_a(src_hbm, dst_hbm, pt_hbm, es_hbm, dpart_hbm,
               sl_v, sr_v, dbuf, sbuf, ebuf, zbuf, denom_sp):
  cid = lax.axis_index("c")
  sid = lax.axis_index("s")
  wid = sid * NC + cid

  pltpu.sync_copy(pt_hbm.at[0], sl_v)
  pltpu.sync_copy(pt_hbm.at[1], sr_v)

  _zero_vmem(zbuf, 640)
  pltpu.sync_copy(zbuf, denom_sp.at[pl.ds(sid * 640, 640)])
  plsc.subcore_barrier()

  @pl.loop(0, MAXT)
  def _(t):
    b = wid + NW * t

    @pl.when(b < NB)
    def _():
      base = b * B
      pltpu.sync_copy(dst_hbm.at[pl.ds(base, B)], dbuf)
      pltpu.sync_copy(src_hbm.at[pl.ds(base, B)], sbuf)
      for g in range(B // 16):
        d16 = dbuf[pl.ds(g * 16, 16)]
        s16 = sbuf[pl.ds(g * 16, 16)]
        sv = plsc.load_gather(sl_v, [d16]) + plsc.load_gather(sr_v, [s16])
        sv = jnp.where(sv > 0, sv, 0.2 * sv)
        ebuf[pl.ds(g * 16, 16)] = jnp.exp(sv)
      pltpu.sync_copy(ebuf, es_hbm.at[pl.ds(base, B)])
      pltpu.sync_copy(ebuf, denom_sp.at[dbuf], add=True)

  plsc.subcore_barrier()
  pltpu.sync_copy(denom_sp.at[pl.ds(sid * 640, 640)],
                  dpart_hbm.at[cid, pl.ds(sid * 640, 640)])


@functools.partial(
    pl.kernel,
    out_type=jax.ShapeDtypeStruct((NC, AGG_PAD), jnp.float32),
    mesh=_mesh,
    scratch_types=[
        pltpu.VMEM((6, N), jnp.float32),
        pltpu.VMEM((NPAD,), jnp.float32),
        pltpu.VMEM((NPAD,), jnp.float32),
        pltpu.VMEM((B,), jnp.int32),
        pltpu.VMEM((B,), jnp.int32),
        pltpu.VMEM((B,), jnp.float32),
        pltpu.VMEM((6 * B,), jnp.float32),
        pltpu.VMEM((6 * B,), jnp.int32),
        pltpu.VMEM_SHARED((AGG_PAD,), jnp.float32),
    ],
    compiler_params=_sc_params,
)
def _sc_pass_b1(src_hbm, dst_hbm, es_hbm, pt_hbm, dpart_hbm,
                aggp_hbm,
                p6_v, dn0, dn1, dbuf, sbuf, ebuf, vbuf, ibuf, aggp_sp):
  cid = lax.axis_index("c")
  sid = lax.axis_index("s")
  wid = sid * NC + cid

  for c in range(6):
    pltpu.sync_copy(pt_hbm.at[2 + c], p6_v.at[c])
  pltpu.sync_copy(dpart_hbm.at[0], dn0)
  pltpu.sync_copy(dpart_hbm.at[1], dn1)

  @pl.loop(0, NPAD, step=16)
  def _(i):
    dn0[pl.ds(i, 16)] = dn0[pl.ds(i, 16)] + dn1[pl.ds(i, 16)] + 1e-16

  _zero_vmem(dn1, 5120)
  pltpu.sync_copy(dn1.at[pl.ds(0, 5120)],
                  aggp_sp.at[pl.ds(sid * 5120, 5120)])
  plsc.subcore_barrier()

  @pl.loop(0, MAXT)
  def _(t):
    b = wid + NW * t

    @pl.when(b < NB)
    def _():
      base = b * B
      pltpu.sync_copy(dst_hbm.at[pl.ds(base, B)], dbuf)
      pltpu.sync_copy(src_hbm.at[pl.ds(base, B)], sbuf)
      pltpu.sync_copy(es_hbm.at[pl.ds(base, B)], ebuf)
      for g in range(B // 16):
        d16 = dbuf[pl.ds(g * 16, 16)]
        s16 = sbuf[pl.ds(g * 16, 16)]
        e16 = ebuf[pl.ds(g * 16, 16)]
        a16 = e16 / plsc.load_gather(dn0, [d16])
        for c in range(6):
          pc = plsc.load_gather(p6_v, [jnp.full((16,), c, jnp.int32), s16])
          vbuf[pl.ds(g * 96 + c * 16, 16)] = a16 * pc
          ibuf[pl.ds(g * 96 + c * 16, 16)] = d16 * 8 + (c + 2)
      pltpu.sync_copy(vbuf, aggp_sp.at[ibuf], add=True)

  plsc.subcore_barrier()
  pltpu.sync_copy(aggp_sp.at[pl.ds(sid * 5120, 5120)],
                  aggp_hbm.at[cid, pl.ds(sid * 5120, 5120)])


@functools.partial(
    pl.kernel,
    out_type=(
        jax.ShapeDtypeStruct((NC, WMAT_PAD), jnp.float32),
        jax.ShapeDtypeStruct((T,), jnp.int32),
    ),
    mesh=_mesh,
    scratch_types=[
        pltpu.VMEM((NPAD,), jnp.float32),
        pltpu.VMEM((NPAD,), jnp.float32),
        pltpu.VMEM((N,), jnp.int32),
        pltpu.VMEM((T,), jnp.int32),
        pltpu.VMEM((T,), jnp.int32),
        pltpu.VMEM((B,), jnp.int32),
        pltpu.VMEM((B,), jnp.int32),
        pltpu.VMEM((B,), jnp.float32),
        pltpu.VMEM((B,), jnp.float32),
        pltpu.VMEM((B,), jnp.int32),
        pltpu.VMEM_SHARED((WMAT_PAD,), jnp.float32),
    ],
    compiler_params=_sc_params,
)
def _sc_pass_b2(src_hbm, dst_hbm, es_hbm, dpart_hbm, clue_hbm,
                wmat_hbm, slotpos_hbm,
                dn0, dn1, slot_v, cluebuf, spbuf,
                dbuf, sbuf, ebuf, abuf, wibuf, wmat_sp):
  cid = lax.axis_index("c")
  sid = lax.axis_index("s")
  wid = sid * NC + cid

  pltpu.sync_copy(dpart_hbm.at[0], dn0)
  pltpu.sync_copy(dpart_hbm.at[1], dn1)

  @pl.loop(0, NPAD, step=16)
  def _(i):
    dn0[pl.ds(i, 16)] = dn0[pl.ds(i, 16)] + dn1[pl.ds(i, 16)] + 1e-16

  pltpu.sync_copy(clue_hbm, cluebuf)
  f64 = jnp.full((16,), 64, jnp.int32)

  @pl.loop(0, N, step=16)
  def _(i):
    slot_v[pl.ds(i, 16)] = f64

  lane0 = lax.iota(jnp.int32, 16) == 0
  for tt in range(T):
    node = plsc.load_gather(cluebuf, [jnp.full((16,), tt, jnp.int32)])
    plsc.store_scatter(slot_v, [node], jnp.full((16,), tt, jnp.int32),
                       mask=lane0)

  _zero_vmem(dn1, NPAD)
  for k in range(4):
    pltpu.sync_copy(dn1, wmat_sp.at[pl.ds(sid * 40960 + k * NPAD, NPAD)])
  plsc.subcore_barrier()

  @pl.when(wid == 0)
  def _():
    for q in range(T // 16):
      c16 = cluebuf[pl.ds(q * 16, 16)]
      spbuf[pl.ds(q * 16, 16)] = plsc.load_gather(slot_v, [c16])
    pltpu.sync_copy(spbuf, slotpos_hbm)

  @pl.loop(0, MAXT)
  def _(t):
    b = wid + NW * t

    @pl.when(b < NB)
    def _():
      base = b * B
      pltpu.sync_copy(dst_hbm.at[pl.ds(base, B)], dbuf)
      pltpu.sync_copy(src_hbm.at[pl.ds(base, B)], sbuf)
      pltpu.sync_copy(es_hbm.at[pl.ds(base, B)], ebuf)
      for g in range(B // 16):
        d16 = dbuf[pl.ds(g * 16, 16)]
        s16 = sbuf[pl.ds(g * 16, 16)]
        e16 = ebuf[pl.ds(g * 16, 16)]
        abuf[pl.ds(g * 16, 16)] = e16 / plsc.load_gather(dn0, [d16])
        st16 = plsc.load_gather(slot_v, [d16])
        wibuf[pl.ds(g * 16, 16)] = st16 * N + s16
      pltpu.sync_copy(abuf, wmat_sp.at[wibuf], add=True)

  plsc.subcore_barrier()
  pltpu.sync_copy(wmat_sp.at[pl.ds(sid * 40960, 40960)],
                  wmat_hbm.at[cid, pl.ds(sid * 40960, 40960)])


def _k1_body(x_ref, wg_ref, bg_ref, wp_ref, wgx_ref, pt_ref, q_ref):
  xb = x_ref[...]
  wgx = lax.dot_general(xb, wg_ref[...], (((1,), (1,)), ((), ())),
                        preferred_element_type=jnp.float32) + bg_ref[...]
  wgx_ref[...] = wgx
  pt_ref[...] = lax.dot_general(wp_ref[...], wgx, (((1,), (1,)), ((), ())),
                                preferred_element_type=jnp.float32)
  q_ref[...] = lax.dot_general(xb, wp_ref[...], (((1,), (1,)), ((), ())),
                               preferred_element_type=jnp.float32)


def _k1(x, W_g, b_g2, Wproj):
  return pl.pallas_call(
      _k1_body,
      out_shape=[
          jax.ShapeDtypeStruct((N, D), jnp.float32),
          jax.ShapeDtypeStruct((8, N), jnp.float32),
          jax.ShapeDtypeStruct((N, 8), jnp.float32),
      ],
  )(x, W_g, b_g2, Wproj)


def _k3_body(x_ref, wgx_ref, w0_ref, w1_ref, p0_ref, p1_ref, q_ref,
             clue_ref, sp_ref, wih_ref, whh_ref, bih_ref, bhh_ref,
             wc2_ref, we2_ref, bc_ref, be_ref,
             oc_ref, oe_ref, aggc_ref, clues_ref, gi_ref):
  wmat = w0_ref[...] + w1_ref[...]
  aggc_ref[...] = lax.dot_general(wmat, wgx_ref[...],
                                  (((1,), (0,)), ((), ())),
                                  preferred_element_type=jnp.float32)

  def build_row(t, _):
    xr = x_ref[pl.ds(clue_ref[t], 1), :]
    cr = aggc_ref[pl.ds(sp_ref[t], 1), :]
    clues_ref[pl.ds(t, 1), :] = xr + cr
    return 0

  lax.fori_loop(0, T, build_row, 0)

  gi_ref[...] = lax.dot_general(clues_ref[...], wih_ref[...],
                                (((1,), (1,)), ((), ())),
                                preferred_element_type=jnp.float32) + bih_ref[...]

  def gru_step(t, h):
    gh = lax.dot_general(h, whh_ref[...], (((1,), (1,)), ((), ())),
                         preferred_element_type=jnp.float32) + bhh_ref[...]
    gi = gi_ref[pl.ds(t, 1), :]
    r = jax.nn.sigmoid(gi[:, :H] + gh[:, :H])
    z = jax.nn.sigmoid(gi[:, H:2 * H] + gh[:, H:2 * H])
    ng = jnp.tanh(gi[:, 2 * H:] + r * gh[:, 2 * H:])
    return (1.0 - z) * ng + z * h

  h = lax.fori_loop(0, T, gru_step, jnp.zeros((1, H), jnp.float32))

  cc = lax.dot_general(h, wc2_ref[...], (((1,), (1,)), ((), ())),
                       preferred_element_type=jnp.float32) + bc_ref[...]
  ec = lax.dot_general(h, we2_ref[...], (((1,), (1,)), ((), ())),
                       preferred_element_type=jnp.float32) + be_ref[...]

  q = q_ref[...]
  p0 = p0_ref[...][:N, :]
  p1 = p1_ref[...][:N, :]
  oc_ref[...] = q[:, 2:5] + p0[:, 2:5] + p1[:, 2:5] + cc
  oe_ref[...] = q[:, 5:8] + p0[:, 5:8] + p1[:, 5:8] + ec


def _k3(x, wgx, w0, w1, p0, p1, q, clue_idx, slotpos,
        W_ih, W_hh, b_ih2, b_hh2, wc2, we2, bc2, be2):
  sspec = pl.BlockSpec(memory_space=pltpu.SMEM)
  return pl.pallas_call(
      _k3_body,
      in_specs=[
          pl.BlockSpec(memory_space=pltpu.VMEM),
          pl.BlockSpec(memory_space=pltpu.VMEM),
          pl.BlockSpec(memory_space=pltpu.VMEM),
          pl.BlockSpec(memory_space=pltpu.VMEM),
          pl.BlockSpec(memory_space=pltpu.VMEM),
          pl.BlockSpec(memory_space=pltpu.VMEM),
          pl.BlockSpec(memory_space=pltpu.VMEM),
          sspec,
          sspec,
          pl.BlockSpec(memory_space=pltpu.VMEM),
          pl.BlockSpec(memory_space=pltpu.VMEM),
          pl.BlockSpec(memory_space=pltpu.VMEM),
          pl.BlockSpec(memory_space=pltpu.VMEM),
          pl.BlockSpec(memory_space=pltpu.VMEM),
          pl.BlockSpec(memory_space=pltpu.VMEM),
          pl.BlockSpec(memory_space=pltpu.VMEM),
          pl.BlockSpec(memory_space=pltpu.VMEM),
      ],
      out_specs=[
          pl.BlockSpec(memory_space=pltpu.VMEM),
          pl.BlockSpec(memory_space=pltpu.VMEM),
      ],
      out_shape=[
          jax.ShapeDtypeStruct((N, 3), jnp.float32),
          jax.ShapeDtypeStruct((N, 3), jnp.float32),
      ],
      scratch_shapes=[
          pltpu.VMEM((T, D), jnp.float32),
          pltpu.VMEM((T, D), jnp.float32),
          pltpu.VMEM((T, 3 * H), jnp.float32),
      ],
  )(x, wgx, w0, w1, p0, p1, q, clue_idx, slotpos,
    W_ih, W_hh, b_ih2, b_hh2, wc2, we2, bc2, be2)


def kernel(x, edge_index, clue_idx, W_g, b_g, alpha_left, alpha_right,
           W_ih, W_hh, b_ih, b_hh, W_cause, b_cause, W_effect, b_effect):
  Wproj = jnp.concatenate(
      [alpha_left[None, :], alpha_right[None, :],
       W_cause[:, :D], W_effect[:, :D]], axis=0)

  wgx, pt, q = _k1(x, W_g, b_g[None, :], Wproj)

  src = edge_index[0]
  dst = edge_index[1]

  es, dparts = _sc_pass_a(src, dst, pt)
  aggp = _sc_pass_b1(src, dst, es, pt, dparts)
  wmatp, slotpos = _sc_pass_b2(src, dst, es, dparts, clue_idx)

  w0 = wmatp[0, :64 * N].reshape(T, N)
  w1 = wmatp[1, :64 * N].reshape(T, N)
  p0 = aggp[0].reshape(NPAD, 8)
  p1 = aggp[1].reshape(NPAD, 8)

  oc, oe = _k3(x, wgx, w0, w1, p0, p1, q, clue_idx, slotpos,
               W_ih, W_hh, b_ih[None, :], b_hh[None, :],
               W_cause[:, D:], W_effect[:, D:],
               b_cause[None, :], b_effect[None, :])
  return oc, oe


# speedup vs baseline: 14.5834x; 1.4789x over previous
import dataclasses
import functools

import jax
import jax.numpy as jnp
from jax import lax
from jax.experimental import pallas as pl
from jax.experimental.pallas import tpu as pltpu
from jax.experimental.pallas import tpu_sc as plsc

N = 10000
E = 160000
D = 256
H = 128
T = 64

NC = 2
NS = 16
NW = NC * NS
B = 640
NB = E // B
MAXT = (NB + NW - 1) // NW

NPAD = 10240
AGG_PAD = 81920
WMAT_PAD = 655360

_mesh = plsc.VectorSubcoreMesh(core_axis_name="c", subcore_axis_name="s")

_sc_params = pltpu.CompilerParams()
if "needs_layout_passes" in pltpu.CompilerParams.__dataclass_fields__:
  _sc_params = dataclasses.replace(_sc_params, needs_layout_passes=False)


def _zero_vmem(ref, n):
  z = jnp.zeros((16,), ref.dtype)

  @pl.loop(0, n, step=16)
  def _(i):
    ref[pl.ds(i, 16)] = z


@functools.partial(
    pl.kernel,
    out_type=(
        jax.ShapeDtypeStruct((E,), jnp.float32),
        jax.ShapeDtypeStruct((NC, NPAD), jnp.float32),
    ),
    mesh=_mesh,
    scratch_types=[
        pltpu.VMEM((N,), jnp.float32),
        pltpu.VMEM((N,), jnp.float32),
        pltpu.VMEM((B,), jnp.int32),
        pltpu.VMEM((B,), jnp.int32),
        pltpu.VMEM((B,), jnp.float32),
        pltpu.VMEM((640,), jnp.float32),
        pltpu.VMEM_SHARED((NPAD,), jnp.float32),
    ],
    compiler_params=_sc_params,
)
def _sc_pass_a(src_hbm, dst_hbm, pt_hbm, es_hbm, dpart_hbm,
               sl_v, sr_v, dbuf, sbuf, ebuf, zbuf, denom_sp):
  cid = lax.axis_index("c")
  sid = lax.axis_index("s")
  wid = sid * NC + cid

  pltpu.sync_copy(pt_hbm.at[0], sl_v)
  pltpu.sync_copy(pt_hbm.at[1], sr_v)

  _zero_vmem(zbuf, 640)
  pltpu.sync_copy(zbuf, denom_sp.at[pl.ds(sid * 640, 640)])
  plsc.subcore_barrier()

  @pl.loop(0, MAXT)
  def _(t):
    b = wid + NW * t

    @pl.when(b < NB)
    def _():
      base = b * B
      pltpu.sync_copy(dst_hbm.at[pl.ds(base, B)], dbuf)
      pltpu.sync_copy(src_hbm.at[pl.ds(base, B)], sbuf)

      @pl.loop(0, B // 16)
      def _(g):
        d16 = dbuf[pl.ds(g * 16, 16)]
        s16 = sbuf[pl.ds(g * 16, 16)]
        sv = plsc.load_gather(sl_v, [d16]) + plsc.load_gather(sr_v, [s16])
        sv = jnp.where(sv > 0, sv, 0.2 * sv)
        ebuf[pl.ds(g * 16, 16)] = jnp.exp(sv)
      pltpu.sync_copy(ebuf, es_hbm.at[pl.ds(base, B)])
      pltpu.sync_copy(ebuf, denom_sp.at[dbuf], add=True)

  plsc.subcore_barrier()
  pltpu.sync_copy(denom_sp.at[pl.ds(sid * 640, 640)],
                  dpart_hbm.at[cid, pl.ds(sid * 640, 640)])


@functools.partial(
    pl.kernel,
    out_type=jax.ShapeDtypeStruct((NC, AGG_PAD), jnp.float32),
    mesh=_mesh,
    scratch_types=[
        pltpu.VMEM((6, N), jnp.float32),
        pltpu.VMEM((NPAD,), jnp.float32),
        pltpu.VMEM((NPAD,), jnp.float32),
        pltpu.VMEM((B,), jnp.int32),
        pltpu.VMEM((B,), jnp.int32),
        pltpu.VMEM((B,), jnp.float32),
        pltpu.VMEM((6 * B,), jnp.float32),
        pltpu.VMEM((6 * B,), jnp.int32),
        pltpu.VMEM_SHARED((AGG_PAD,), jnp.float32),
    ],
    compiler_params=_sc_params,
)
def _sc_pass_b1(src_hbm, dst_hbm, es_hbm, pt_hbm, dpart_hbm,
                aggp_hbm,
                p6_v, dn0, dn1, dbuf, sbuf, ebuf, vbuf, ibuf, aggp_sp):
  cid = lax.axis_index("c")
  sid = lax.axis_index("s")
  wid = sid * NC + cid

  for c in range(6):
    pltpu.sync_copy(pt_hbm.at[2 + c], p6_v.at[c])
  pltpu.sync_copy(dpart_hbm.at[0], dn0)
  pltpu.sync_copy(dpart_hbm.at[1], dn1)

  @pl.loop(0, NPAD, step=16)
  def _(i):
    dn0[pl.ds(i, 16)] = dn0[pl.ds(i, 16)] + dn1[pl.ds(i, 16)] + 1e-16

  _zero_vmem(dn1, 5120)
  pltpu.sync_copy(dn1.at[pl.ds(0, 5120)],
                  aggp_sp.at[pl.ds(sid * 5120, 5120)])
  plsc.subcore_barrier()

  @pl.loop(0, MAXT)
  def _(t):
    b = wid + NW * t

    @pl.when(b < NB)
    def _():
      base = b * B
      pltpu.sync_copy(dst_hbm.at[pl.ds(base, B)], dbuf)
      pltpu.sync_copy(src_hbm.at[pl.ds(base, B)], sbuf)
      pltpu.sync_copy(es_hbm.at[pl.ds(base, B)], ebuf)

      @pl.loop(0, B // 16)
      def _(g):
        d16 = dbuf[pl.ds(g * 16, 16)]
        s16 = sbuf[pl.ds(g * 16, 16)]
        e16 = ebuf[pl.ds(g * 16, 16)]
        a16 = e16 / plsc.load_gather(dn0, [d16])
        for c in range(6):
          pc = plsc.load_gather(p6_v, [jnp.full((16,), c, jnp.int32), s16])
          vbuf[pl.ds(g * 96 + c * 16, 16)] = a16 * pc
          ibuf[pl.ds(g * 96 + c * 16, 16)] = d16 * 8 + (c + 2)
      pltpu.sync_copy(vbuf, aggp_sp.at[ibuf], add=True)

  plsc.subcore_barrier()
  pltpu.sync_copy(aggp_sp.at[pl.ds(sid * 5120, 5120)],
                  aggp_hbm.at[cid, pl.ds(sid * 5120, 5120)])


@functools.partial(
    pl.kernel,
    out_type=(
        jax.ShapeDtypeStruct((NC, WMAT_PAD), jnp.float32),
        jax.ShapeDtypeStruct((T,), jnp.int32),
    ),
    mesh=_mesh,
    scratch_types=[
        pltpu.VMEM((NPAD,), jnp.float32),
        pltpu.VMEM((NPAD,), jnp.float32),
        pltpu.VMEM((N,), jnp.int32),
        pltpu.VMEM((T,), jnp.int32),
        pltpu.VMEM((T,), jnp.int32),
        pltpu.VMEM((B,), jnp.int32),
        pltpu.VMEM((B,), jnp.int32),
        pltpu.VMEM((B,), jnp.float32),
        pltpu.VMEM((B,), jnp.float32),
        pltpu.VMEM((B,), jnp.int32),
        pltpu.VMEM_SHARED((WMAT_PAD,), jnp.float32),
    ],
    compiler_params=_sc_params,
)
def _sc_pass_b2(src_hbm, dst_hbm, es_hbm, dpart_hbm, clue_hbm,
                wmat_hbm, slotpos_hbm,
                dn0, dn1, slot_v, cluebuf, spbuf,
                dbuf, sbuf, ebuf, abuf, wibuf, wmat_sp):
  cid = lax.axis_index("c")
  sid = lax.axis_index("s")
  wid = sid * NC + cid

  pltpu.sync_copy(dpart_hbm.at[0], dn0)
  pltpu.sync_copy(dpart_hbm.at[1], dn1)

  @pl.loop(0, NPAD, step=16)
  def _(i):
    dn0[pl.ds(i, 16)] = dn0[pl.ds(i, 16)] + dn1[pl.ds(i, 16)] + 1e-16

  pltpu.sync_copy(clue_hbm, cluebuf)
  f64 = jnp.full((16,), 64, jnp.int32)

  @pl.loop(0, N, step=16)
  def _(i):
    slot_v[pl.ds(i, 16)] = f64

  lane0 = lax.iota(jnp.int32, 16) == 0
  for tt in range(T):
    node = plsc.load_gather(cluebuf, [jnp.full((16,), tt, jnp.int32)])
    plsc.store_scatter(slot_v, [node], jnp.full((16,), tt, jnp.int32),
                       mask=lane0)

  _zero_vmem(dn1, NPAD)
  for k in range(4):
    pltpu.sync_copy(dn1, wmat_sp.at[pl.ds(sid * 40960 + k * NPAD, NPAD)])
  plsc.subcore_barrier()

  @pl.when(wid == 0)
  def _():
    for q in range(T // 16):
      c16 = cluebuf[pl.ds(q * 16, 16)]
      spbuf[pl.ds(q * 16, 16)] = plsc.load_gather(slot_v, [c16])
    pltpu.sync_copy(spbuf, slotpos_hbm)

  @pl.loop(0, MAXT)
  def _(t):
    b = wid + NW * t

    @pl.when(b < NB)
    def _():
      base = b * B
      pltpu.sync_copy(dst_hbm.at[pl.ds(base, B)], dbuf)
      pltpu.sync_copy(src_hbm.at[pl.ds(base, B)], sbuf)
      pltpu.sync_copy(es_hbm.at[pl.ds(base, B)], ebuf)

      @pl.loop(0, B // 16)
      def _(g):
        d16 = dbuf[pl.ds(g * 16, 16)]
        s16 = sbuf[pl.ds(g * 16, 16)]
        e16 = ebuf[pl.ds(g * 16, 16)]
        abuf[pl.ds(g * 16, 16)] = e16 / plsc.load_gather(dn0, [d16])
        st16 = plsc.load_gather(slot_v, [d16])
        wibuf[pl.ds(g * 16, 16)] = st16 * N + s16
      pltpu.sync_copy(abuf, wmat_sp.at[wibuf], add=True)

  plsc.subcore_barrier()
  pltpu.sync_copy(wmat_sp.at[pl.ds(sid * 40960, 40960)],
                  wmat_hbm.at[cid, pl.ds(sid * 40960, 40960)])


def _k1_body(x_ref, wg_ref, bg_ref, wp_ref, wgx_ref, pt_ref, q_ref):
  xb = x_ref[...]
  wgx = lax.dot_general(xb, wg_ref[...], (((1,), (1,)), ((), ())),
                        preferred_element_type=jnp.float32) + bg_ref[...]
  wgx_ref[...] = wgx
  pt_ref[...] = lax.dot_general(wp_ref[...], wgx, (((1,), (1,)), ((), ())),
                                preferred_element_type=jnp.float32)
  q_ref[...] = lax.dot_general(xb, wp_ref[...], (((1,), (1,)), ((), ())),
                               preferred_element_type=jnp.float32)


def _k1(x, W_g, b_g2, Wproj):
  return pl.pallas_call(
      _k1_body,
      out_shape=[
          jax.ShapeDtypeStruct((N, D), jnp.float32),
          jax.ShapeDtypeStruct((8, N), jnp.float32),
          jax.ShapeDtypeStruct((N, 8), jnp.float32),
      ],
  )(x, W_g, b_g2, Wproj)


def _k3_body(x_ref, wgx_ref, w0_ref, w1_ref, p0_ref, p1_ref, q_ref,
             clue_ref, sp_ref, wih_ref, whh_ref, bih_ref, bhh_ref,
             wc2_ref, we2_ref, bc_ref, be_ref,
             oc_ref, oe_ref, aggc_ref, clues_ref, gi_ref):
  wmat = w0_ref[...] + w1_ref[...]
  aggc_ref[...] = lax.dot_general(wmat, wgx_ref[...],
                                  (((1,), (0,)), ((), ())),
                                  preferred_element_type=jnp.float32)

  def build_row(t, _):
    xr = x_ref[pl.ds(clue_ref[t], 1), :]
    cr = aggc_ref[pl.ds(sp_ref[t], 1), :]
    clues_ref[pl.ds(t, 1), :] = xr + cr
    return 0

  lax.fori_loop(0, T, build_row, 0)

  gi_ref[...] = lax.dot_general(clues_ref[...], wih_ref[...],
                                (((1,), (1,)), ((), ())),
                                preferred_element_type=jnp.float32) + bih_ref[...]

  def gru_step(t, h):
    gh = lax.dot_general(h, whh_ref[...], (((1,), (1,)), ((), ())),
                         preferred_element_type=jnp.float32) + bhh_ref[...]
    gi = gi_ref[pl.ds(t, 1), :]
    r = jax.nn.sigmoid(gi[:, :H] + gh[:, :H])
    z = jax.nn.sigmoid(gi[:, H:2 * H] + gh[:, H:2 * H])
    ng = jnp.tanh(gi[:, 2 * H:] + r * gh[:, 2 * H:])
    return (1.0 - z) * ng + z * h

  h = lax.fori_loop(0, T, gru_step, jnp.zeros((1, H), jnp.float32))

  cc = lax.dot_general(h, wc2_ref[...], (((1,), (1,)), ((), ())),
                       preferred_element_type=jnp.float32) + bc_ref[...]
  ec = lax.dot_general(h, we2_ref[...], (((1,), (1,)), ((), ())),
                       preferred_element_type=jnp.float32) + be_ref[...]

  q = q_ref[...]
  p0 = p0_ref[...][:N, :]
  p1 = p1_ref[...][:N, :]
  oc_ref[...] = q[:, 2:5] + p0[:, 2:5] + p1[:, 2:5] + cc
  oe_ref[...] = q[:, 5:8] + p0[:, 5:8] + p1[:, 5:8] + ec


def _k3(x, wgx, w0, w1, p0, p1, q, clue_idx, slotpos,
        W_ih, W_hh, b_ih2, b_hh2, wc2, we2, bc2, be2):
  sspec = pl.BlockSpec(memory_space=pltpu.SMEM)
  return pl.pallas_call(
      _k3_body,
      in_specs=[
          pl.BlockSpec(memory_space=pltpu.VMEM),
          pl.BlockSpec(memory_space=pltpu.VMEM),
          pl.BlockSpec(memory_space=pltpu.VMEM),
          pl.BlockSpec(memory_space=pltpu.VMEM),
          pl.BlockSpec(memory_space=pltpu.VMEM),
          pl.BlockSpec(memory_space=pltpu.VMEM),
          pl.BlockSpec(memory_space=pltpu.VMEM),
          sspec,
          sspec,
          pl.BlockSpec(memory_space=pltpu.VMEM),
          pl.BlockSpec(memory_space=pltpu.VMEM),
          pl.BlockSpec(memory_space=pltpu.VMEM),
          pl.BlockSpec(memory_space=pltpu.VMEM),
          pl.BlockSpec(memory_space=pltpu.VMEM),
          pl.BlockSpec(memory_space=pltpu.VMEM),
          pl.BlockSpec(memory_space=pltpu.VMEM),
          pl.BlockSpec(memory_space=pltpu.VMEM),
      ],
      out_specs=[
          pl.BlockSpec(memory_space=pltpu.VMEM),
          pl.BlockSpec(memory_space=pltpu.VMEM),
      ],
      out_shape=[
          jax.ShapeDtypeStruct((N, 3), jnp.float32),
          jax.ShapeDtypeStruct((N, 3), jnp.float32),
      ],
      scratch_shapes=[
          pltpu.VMEM((T, D), jnp.float32),
          pltpu.VMEM((T, D), jnp.float32),
          pltpu.VMEM((T, 3 * H), jnp.float32),
      ],
  )(x, wgx, w0, w1, p0, p1, q, clue_idx, slotpos,
    W_ih, W_hh, b_ih2, b_hh2, wc2, we2, bc2, be2)


def kernel(x, edge_index, clue_idx, W_g, b_g, alpha_left, alpha_right,
           W_ih, W_hh, b_ih, b_hh, W_cause, b_cause, W_effect, b_effect):
  Wproj = jnp.concatenate(
      [alpha_left[None, :], alpha_right[None, :],
       W_cause[:, :D], W_effect[:, :D]], axis=0)

  wgx, pt, q = _k1(x, W_g, b_g[None, :], Wproj)

  src = edge_index[0]
  dst = edge_index[1]

  es, dparts = _sc_pass_a(src, dst, pt)
  aggp = _sc_pass_b1(src, dst, es, pt, dparts)
  wmatp, slotpos = _sc_pass_b2(src, dst, es, dparts, clue_idx)

  w0 = wmatp[0, :64 * N].reshape(T, N)
  w1 = wmatp[1, :64 * N].reshape(T, N)
  p0 = aggp[0].reshape(NPAD, 8)
  p1 = aggp[1].reshape(NPAD, 8)

  oc, oe = _k3(x, wgx, w0, w1, p0, p1, q, clue_idx, slotpos,
               W_ih, W_hh, b_ih[None, :], b_hh[None, :],
               W_cause[:, D:], W_effect[:, D:],
               b_cause[None, :], b_effect[None, :])
  return oc, oe


# speedup vs baseline: 16.3387x; 1.1204x over previous
import dataclasses
import functools

import jax
import jax.numpy as jnp
from jax import lax
from jax.experimental import pallas as pl
from jax.experimental.pallas import tpu as pltpu
from jax.experimental.pallas import tpu_sc as plsc

N = 10000
E = 160000
D = 256
H = 128
T = 64

NC = 2
NS = 16
NW = NC * NS
B = 640
NB = E // B
MAXT = (NB + NW - 1) // NW

NPAD = 10240
AGG_PAD = 81920
WMAT_PAD = 665600

_mesh = plsc.VectorSubcoreMesh(core_axis_name="c", subcore_axis_name="s")

_sc_params = pltpu.CompilerParams()
if "needs_layout_passes" in pltpu.CompilerParams.__dataclass_fields__:
  _sc_params = dataclasses.replace(_sc_params, needs_layout_passes=False)


def _zero_vmem(ref, n):
  z = jnp.zeros((16,), ref.dtype)

  @pl.loop(0, n, step=16)
  def _(i):
    ref[pl.ds(i, 16)] = z


@functools.partial(
    pl.kernel,
    out_type=(
        jax.ShapeDtypeStruct((E,), jnp.float32),
        jax.ShapeDtypeStruct((NC, NPAD), jnp.float32),
    ),
    mesh=_mesh,
    scratch_types=[
        pltpu.VMEM((N,), jnp.float32),
        pltpu.VMEM((N,), jnp.float32),
        pltpu.VMEM((B,), jnp.int32),
        pltpu.VMEM((B,), jnp.int32),
        pltpu.VMEM((B,), jnp.float32),
        pltpu.VMEM((640,), jnp.float32),
        pltpu.VMEM_SHARED((NPAD,), jnp.float32),
    ],
    compiler_params=_sc_params,
)
def _sc_pass_a(ei_hbm, pt_hbm, es_hbm, dpart_hbm,
               sl_v, sr_v, dbuf, sbuf, ebuf, zbuf, denom_sp):
  cid = lax.axis_index("c")
  sid = lax.axis_index("s")
  wid = sid * NC + cid

  pltpu.sync_copy(pt_hbm.at[0], sl_v)
  pltpu.sync_copy(pt_hbm.at[1], sr_v)

  _zero_vmem(zbuf, 640)
  pltpu.sync_copy(zbuf, denom_sp.at[pl.ds(sid * 640, 640)])
  plsc.subcore_barrier()

  @pl.loop(0, MAXT)
  def _(t):
    b = wid + NW * t

    @pl.when(b < NB)
    def _():
      base = b * B
      pltpu.sync_copy(ei_hbm.at[pl.ds(E + base, B)], dbuf)
      pltpu.sync_copy(ei_hbm.at[pl.ds(base, B)], sbuf)

      @pl.loop(0, B // 16)
      def _(g):
        d16 = dbuf[pl.ds(g * 16, 16)]
        s16 = sbuf[pl.ds(g * 16, 16)]
        sv = plsc.load_gather(sl_v, [d16]) + plsc.load_gather(sr_v, [s16])
        sv = jnp.where(sv > 0, sv, 0.2 * sv)
        ebuf[pl.ds(g * 16, 16)] = jnp.exp(sv)
      pltpu.sync_copy(ebuf, es_hbm.at[pl.ds(base, B)])
      pltpu.sync_copy(ebuf, denom_sp.at[dbuf], add=True)

  plsc.subcore_barrier()
  pltpu.sync_copy(denom_sp.at[pl.ds(sid * 640, 640)],
                  dpart_hbm.at[cid, pl.ds(sid * 640, 640)])


@functools.partial(
    pl.kernel,
    out_type=jax.ShapeDtypeStruct((NC, AGG_PAD), jnp.float32),
    mesh=_mesh,
    scratch_types=[
        pltpu.VMEM((6, N), jnp.float32),
        pltpu.VMEM((NPAD,), jnp.float32),
        pltpu.VMEM((NPAD,), jnp.float32),
        pltpu.VMEM((B,), jnp.int32),
        pltpu.VMEM((B,), jnp.int32),
        pltpu.VMEM((B,), jnp.float32),
        pltpu.VMEM((6 * B,), jnp.float32),
        pltpu.VMEM((6 * B,), jnp.int32),
        pltpu.VMEM_SHARED((AGG_PAD,), jnp.float32),
    ],
    compiler_params=_sc_params,
)
def _sc_pass_b1(ei_hbm, es_hbm, pt_hbm, dpart_hbm,
                aggp_hbm,
                p6_v, dn0, dn1, dbuf, sbuf, ebuf, vbuf, ibuf, aggp_sp):
  cid = lax.axis_index("c")
  sid = lax.axis_index("s")
  wid = sid * NC + cid

  for c in range(6):
    pltpu.sync_copy(pt_hbm.at[2 + c], p6_v.at[c])
  pltpu.sync_copy(dpart_hbm.at[0], dn0)
  pltpu.sync_copy(dpart_hbm.at[1], dn1)

  @pl.loop(0, NPAD, step=16)
  def _(i):
    dn0[pl.ds(i, 16)] = dn0[pl.ds(i, 16)] + dn1[pl.ds(i, 16)] + 1e-16

  _zero_vmem(dn1, 5120)
  pltpu.sync_copy(dn1.at[pl.ds(0, 5120)],
                  aggp_sp.at[pl.ds(sid * 5120, 5120)])
  plsc.subcore_barrier()

  @pl.loop(0, MAXT)
  def _(t):
    b = wid + NW * t

    @pl.when(b < NB)
    def _():
      base = b * B
      pltpu.sync_copy(ei_hbm.at[pl.ds(E + base, B)], dbuf)
      pltpu.sync_copy(ei_hbm.at[pl.ds(base, B)], sbuf)
      pltpu.sync_copy(es_hbm.at[pl.ds(base, B)], ebuf)

      @pl.loop(0, B // 16)
      def _(g):
        d16 = dbuf[pl.ds(g * 16, 16)]
        s16 = sbuf[pl.ds(g * 16, 16)]
        e16 = ebuf[pl.ds(g * 16, 16)]
        a16 = e16 / plsc.load_gather(dn0, [d16])
        for c in range(6):
          pc = plsc.load_gather(p6_v, [jnp.full((16,), c, jnp.int32), s16])
          vbuf[pl.ds(g * 96 + c * 16, 16)] = a16 * pc
          ibuf[pl.ds(g * 96 + c * 16, 16)] = d16 * 8 + (c + 2)
      pltpu.sync_copy(vbuf, aggp_sp.at[ibuf], add=True)

  plsc.subcore_barrier()
  pltpu.sync_copy(aggp_sp.at[pl.ds(sid * 5120, 5120)],
                  aggp_hbm.at[cid, pl.ds(sid * 5120, 5120)])


@functools.partial(
    pl.kernel,
    out_type=(
        jax.ShapeDtypeStruct((NC, T * NPAD), jnp.float32),
        jax.ShapeDtypeStruct((T,), jnp.int32),
    ),
    mesh=_mesh,
    scratch_types=[
        pltpu.VMEM((NPAD,), jnp.float32),
        pltpu.VMEM((NPAD,), jnp.float32),
        pltpu.VMEM((N,), jnp.int32),
        pltpu.VMEM((T,), jnp.int32),
        pltpu.VMEM((T,), jnp.int32),
        pltpu.VMEM((B,), jnp.int32),
        pltpu.VMEM((B,), jnp.int32),
        pltpu.VMEM((B,), jnp.float32),
        pltpu.VMEM((B,), jnp.float32),
        pltpu.VMEM((B,), jnp.int32),
        pltpu.VMEM_SHARED((WMAT_PAD,), jnp.float32),
    ],
    compiler_params=_sc_params,
)
def _sc_pass_b2(ei_hbm, es_hbm, dpart_hbm, clue_hbm,
                wmat_hbm, slotpos_hbm,
                dn0, dn1, slot_v, cluebuf, spbuf,
                dbuf, sbuf, ebuf, abuf, wibuf, wmat_sp):
  cid = lax.axis_index("c")
  sid = lax.axis_index("s")
  wid = sid * NC + cid

  pltpu.sync_copy(dpart_hbm.at[0], dn0)
  pltpu.sync_copy(dpart_hbm.at[1], dn1)

  @pl.loop(0, NPAD, step=16)
  def _(i):
    dn0[pl.ds(i, 16)] = dn0[pl.ds(i, 16)] + dn1[pl.ds(i, 16)] + 1e-16

  pltpu.sync_copy(clue_hbm, cluebuf)
  f64 = jnp.full((16,), 64, jnp.int32)

  @pl.loop(0, N, step=16)
  def _(i):
    slot_v[pl.ds(i, 16)] = f64

  lane0 = lax.iota(jnp.int32, 16) == 0
  for tt in range(T):
    node = plsc.load_gather(cluebuf, [jnp.full((16,), tt, jnp.int32)])
    plsc.store_scatter(slot_v, [node], jnp.full((16,), tt, jnp.int32),
                       mask=lane0)

  _zero_vmem(dn1, NPAD)
  for k in range(4):
    pltpu.sync_copy(dn1, wmat_sp.at[pl.ds(sid * 41600 + k * NPAD, NPAD)])
  pltpu.sync_copy(dn1.at[pl.ds(0, 640)],
                  wmat_sp.at[pl.ds(sid * 41600 + 4 * NPAD, 640)])
  plsc.subcore_barrier()

  @pl.when(wid == 0)
  def _():
    for q in range(T // 16):
      c16 = cluebuf[pl.ds(q * 16, 16)]
      spbuf[pl.ds(q * 16, 16)] = plsc.load_gather(slot_v, [c16])
    pltpu.sync_copy(spbuf, slotpos_hbm)

  @pl.loop(0, MAXT)
  def _(t):
    b = wid + NW * t

    @pl.when(b < NB)
    def _():
      base = b * B
      pltpu.sync_copy(ei_hbm.at[pl.ds(E + base, B)], dbuf)
      pltpu.sync_copy(ei_hbm.at[pl.ds(base, B)], sbuf)
      pltpu.sync_copy(es_hbm.at[pl.ds(base, B)], ebuf)

      @pl.loop(0, B // 16)
      def _(g):
        d16 = dbuf[pl.ds(g * 16, 16)]
        s16 = sbuf[pl.ds(g * 16, 16)]
        e16 = ebuf[pl.ds(g * 16, 16)]
        abuf[pl.ds(g * 16, 16)] = e16 / plsc.load_gather(dn0, [d16])
        st16 = plsc.load_gather(slot_v, [d16])
        wibuf[pl.ds(g * 16, 16)] = st16 * NPAD + s16
      pltpu.sync_copy(abuf, wmat_sp.at[wibuf], add=True)

  plsc.subcore_barrier()
  pltpu.sync_copy(wmat_sp.at[pl.ds(sid * 40960, 40960)],
                  wmat_hbm.at[cid, pl.ds(sid * 40960, 40960)])


def _k1_body(x_ref, wg_ref, bg_ref, wp_ref, wgx_ref, pt_ref, q_ref):
  xb = x_ref[...]
  wgx = lax.dot_general(xb, wg_ref[...], (((1,), (1,)), ((), ())),
                        preferred_element_type=jnp.float32) + bg_ref[...]
  wgx_ref[pl.ds(0, N), :] = wgx
  wgx_ref[pl.ds(N, NPAD - N), :] = jnp.zeros((NPAD - N, D), jnp.float32)
  pt_ref[...] = lax.dot_general(wp_ref[...], wgx, (((1,), (1,)), ((), ())),
                                preferred_element_type=jnp.float32)
  q_ref[...] = lax.dot_general(xb, wp_ref[...], (((1,), (1,)), ((), ())),
                               preferred_element_type=jnp.float32)


def _k1(x, W_g, b_g2, Wproj):
  return pl.pallas_call(
      _k1_body,
      out_shape=[
          jax.ShapeDtypeStruct((NPAD, D), jnp.float32),
          jax.ShapeDtypeStruct((8, N), jnp.float32),
          jax.ShapeDtypeStruct((N, 8), jnp.float32),
      ],
  )(x, W_g, b_g2, Wproj)


def _k3_body(x_ref, wgx_ref, w_ref, p_ref, q_ref,
             clue_ref, sp_ref, wih_ref, whh_ref, bih_ref, bhh_ref,
             wc2_ref, we2_ref, bc_ref, be_ref,
             oc_ref, oe_ref, aggc_ref, clues_ref, gi_ref):
  wmat = w_ref[0] + w_ref[1]
  aggc_ref[...] = lax.dot_general(wmat, wgx_ref[...],
                                  (((1,), (0,)), ((), ())),
                                  preferred_element_type=jnp.float32)

  def build_row(t, _):
    xr = x_ref[pl.ds(clue_ref[t], 1), :]
    cr = aggc_ref[pl.ds(sp_ref[t], 1), :]
    clues_ref[pl.ds(t, 1), :] = xr + cr
    return 0

  lax.fori_loop(0, T, build_row, 0)

  gi_ref[...] = lax.dot_general(clues_ref[...], wih_ref[...],
                                (((1,), (1,)), ((), ())),
                                preferred_element_type=jnp.float32) + bih_ref[...]

  def gru_step(t, h):
    gh = lax.dot_general(h, whh_ref[...], (((1,), (1,)), ((), ())),
                         preferred_element_type=jnp.float32) + bhh_ref[...]
    gi = gi_ref[pl.ds(t, 1), :]
    r = jax.nn.sigmoid(gi[:, :H] + gh[:, :H])
    z = jax.nn.sigmoid(gi[:, H:2 * H] + gh[:, H:2 * H])
    ng = jnp.tanh(gi[:, 2 * H:] + r * gh[:, 2 * H:])
    return (1.0 - z) * ng + z * h

  h = lax.fori_loop(0, T, gru_step, jnp.zeros((1, H), jnp.float32))

  cc = lax.dot_general(h, wc2_ref[...], (((1,), (1,)), ((), ())),
                       preferred_element_type=jnp.float32) + bc_ref[...]
  ec = lax.dot_general(h, we2_ref[...], (((1,), (1,)), ((), ())),
                       preferred_element_type=jnp.float32) + be_ref[...]

  q = q_ref[...]
  p0 = p_ref[0][:N, :]
  p1 = p_ref[1][:N, :]
  oc_ref[...] = q[:, 2:5] + p0[:, 2:5] + p1[:, 2:5] + cc
  oe_ref[...] = q[:, 5:8] + p0[:, 5:8] + p1[:, 5:8] + ec


def _k3(x, wgx, w, p, q, clue_idx, slotpos,
        W_ih, W_hh, b_ih2, b_hh2, wc2, we2, bc2, be2):
  sspec = pl.BlockSpec(memory_space=pltpu.SMEM)
  return pl.pallas_call(
      _k3_body,
      in_specs=[
          pl.BlockSpec(memory_space=pltpu.VMEM),
          pl.BlockSpec(memory_space=pltpu.VMEM),
          pl.BlockSpec(memory_space=pltpu.VMEM),
          pl.BlockSpec(memory_space=pltpu.VMEM),
          pl.BlockSpec(memory_space=pltpu.VMEM),
          sspec,
          sspec,
          pl.BlockSpec(memory_space=pltpu.VMEM),
          pl.BlockSpec(memory_space=pltpu.VMEM),
          pl.BlockSpec(memory_space=pltpu.VMEM),
          pl.BlockSpec(memory_space=pltpu.VMEM),
          pl.BlockSpec(memory_space=pltpu.VMEM),
          pl.BlockSpec(memory_space=pltpu.VMEM),
          pl.BlockSpec(memory_space=pltpu.VMEM),
          pl.BlockSpec(memory_space=pltpu.VMEM),
      ],
      out_specs=[
          pl.BlockSpec(memory_space=pltpu.VMEM),
          pl.BlockSpec(memory_space=pltpu.VMEM),
      ],
      out_shape=[
          jax.ShapeDtypeStruct((N, 3), jnp.float32),
          jax.ShapeDtypeStruct((N, 3), jnp.float32),
      ],
      scratch_shapes=[
          pltpu.VMEM((T, D), jnp.float32),
          pltpu.VMEM((T, D), jnp.float32),
          pltpu.VMEM((T, 3 * H), jnp.float32),
      ],
  )(x, wgx, w, p, q, clue_idx, slotpos,
    W_ih, W_hh, b_ih2, b_hh2, wc2, we2, bc2, be2)


def kernel(x, edge_index, clue_idx, W_g, b_g, alpha_left, alpha_right,
           W_ih, W_hh, b_ih, b_hh, W_cause, b_cause, W_effect, b_effect):
  Wproj = jnp.concatenate(
      [alpha_left[None, :], alpha_right[None, :],
       W_cause[:, :D], W_effect[:, :D]], axis=0)

  wgx, pt, q = _k1(x, W_g, b_g[None, :], Wproj)

  ei = edge_index.reshape(2 * E)
  es, dparts = _sc_pass_a(ei, pt)
  aggp = _sc_pass_b1(ei, es, pt, dparts)
  wmatp, slotpos = _sc_pass_b2(ei, es, dparts, clue_idx)

  w = wmatp.reshape(NC, T, NPAD)
  p = aggp.reshape(NC, NPAD, 8)

  oc, oe = _k3(x, wgx, w, p, q, clue_idx, slotpos,
               W_ih, W_hh, b_ih[None, :], b_hh[None, :],
               W_cause[:, D:], W_effect[:, D:],
               b_cause[None, :], b_effect[None, :])
  return oc, oe


# speedup vs baseline: 18.8653x; 1.1546x over previous
import dataclasses
import functools

import jax
import jax.numpy as jnp
from jax import lax
from jax.experimental import pallas as pl
from jax.experimental.pallas import tpu as pltpu
from jax.experimental.pallas import tpu_sc as plsc

N = 10000
E = 160000
D = 256
H = 128
T = 64

NC = 2
NS = 16
NW = NC * NS
B = 640
NB = E // B
MAXT = (NB + NW - 1) // NW

NPAD = 10240
AGG_PAD = 81920
WMAT_PAD = 665600

_mesh = plsc.VectorSubcoreMesh(core_axis_name="c", subcore_axis_name="s")

_sc_params = pltpu.CompilerParams()
if "needs_layout_passes" in pltpu.CompilerParams.__dataclass_fields__:
  _sc_params = dataclasses.replace(_sc_params, needs_layout_passes=False)


def _zero_vmem(ref, n):
  z = jnp.zeros((16,), ref.dtype)

  @pl.loop(0, n, step=16)
  def _(i):
    ref[pl.ds(i, 16)] = z


@functools.partial(
    pl.kernel,
    out_type=(
        jax.ShapeDtypeStruct((E,), jnp.float32),
        jax.ShapeDtypeStruct((NC, NPAD), jnp.float32),
    ),
    mesh=_mesh,
    scratch_types=[
        pltpu.VMEM((N,), jnp.float32),
        pltpu.VMEM((N,), jnp.float32),
        pltpu.VMEM((B,), jnp.int32),
        pltpu.VMEM((B,), jnp.int32),
        pltpu.VMEM((B,), jnp.int32),
        pltpu.VMEM((B,), jnp.int32),
        pltpu.VMEM((B,), jnp.float32),
        pltpu.VMEM((640,), jnp.float32),
        pltpu.VMEM_SHARED((NPAD,), jnp.float32),
        pltpu.SemaphoreType.DMA,
        pltpu.SemaphoreType.DMA,
    ],
    compiler_params=_sc_params,
)
def _sc_pass_a(ei_hbm, pt_hbm, es_hbm, dpart_hbm,
               sl_v, sr_v, dbuf0, dbuf1, sbuf0, sbuf1, ebuf, zbuf, denom_sp,
               sem0, sem1):
  cid = lax.axis_index("c")
  sid = lax.axis_index("s")
  wid = sid * NC + cid

  pltpu.sync_copy(pt_hbm.at[0], sl_v)
  pltpu.sync_copy(pt_hbm.at[1], sr_v)

  _zero_vmem(zbuf, 640)
  pltpu.sync_copy(zbuf, denom_sp.at[pl.ds(sid * 640, 640)])
  plsc.subcore_barrier()

  sems = (sem0, sem1)
  dbufs = (dbuf0, dbuf1)
  sbufs = (sbuf0, sbuf1)

  def _issue_a(t):
    b = jnp.minimum(wid + NW * t, NB - 1)
    base = b * B
    p = t % 2
    hd = pltpu.async_copy(ei_hbm.at[pl.ds(E + base, B)], dbufs[p], sems[p])
    hs = pltpu.async_copy(ei_hbm.at[pl.ds(base, B)], sbufs[p], sems[p])
    return hd, hs

  hh = _issue_a(0)
  for t in range(MAXT):
    b = wid + NW * t
    for h in hh:
      h.wait()
    if t + 1 < MAXT:
      hh = _issue_a(t + 1)
    dbp = dbufs[t % 2]
    sbp = sbufs[t % 2]

    @pl.when(b < NB)
    def _():
      base = b * B

      @pl.loop(0, B // 16)
      def _(g):
        d16 = dbp[pl.ds(g * 16, 16)]
        s16 = sbp[pl.ds(g * 16, 16)]
        sv = plsc.load_gather(sl_v, [d16]) + plsc.load_gather(sr_v, [s16])
        sv = jnp.where(sv > 0, sv, 0.2 * sv)
        ebuf[pl.ds(g * 16, 16)] = jnp.exp(sv)
      pltpu.sync_copy(ebuf, es_hbm.at[pl.ds(base, B)])
      pltpu.sync_copy(ebuf, denom_sp.at[dbp], add=True)

  plsc.subcore_barrier()
  pltpu.sync_copy(denom_sp.at[pl.ds(sid * 640, 640)],
                  dpart_hbm.at[cid, pl.ds(sid * 640, 640)])


@functools.partial(
    pl.kernel,
    out_type=jax.ShapeDtypeStruct((NC, AGG_PAD), jnp.float32),
    mesh=_mesh,
    scratch_types=[
        pltpu.VMEM((6, N), jnp.float32),
        pltpu.VMEM((NPAD,), jnp.float32),
        pltpu.VMEM((NPAD,), jnp.float32),
        pltpu.VMEM((B,), jnp.int32),
        pltpu.VMEM((B,), jnp.int32),
        pltpu.VMEM((B,), jnp.int32),
        pltpu.VMEM((B,), jnp.int32),
        pltpu.VMEM((B,), jnp.float32),
        pltpu.VMEM((B,), jnp.float32),
        pltpu.VMEM((6 * B,), jnp.float32),
        pltpu.VMEM((6 * B,), jnp.int32),
        pltpu.VMEM_SHARED((AGG_PAD,), jnp.float32),
        pltpu.SemaphoreType.DMA,
        pltpu.SemaphoreType.DMA,
    ],
    compiler_params=_sc_params,
)
def _sc_pass_b1(ei_hbm, es_hbm, pt_hbm, dpart_hbm,
                aggp_hbm,
                p6_v, dn0, dn1, dbuf0, dbuf1, sbuf0, sbuf1, ebuf0, ebuf1,
                vbuf, ibuf, aggp_sp, sem0, sem1):
  cid = lax.axis_index("c")
  sid = lax.axis_index("s")
  wid = sid * NC + cid

  for c in range(6):
    pltpu.sync_copy(pt_hbm.at[2 + c], p6_v.at[c])
  pltpu.sync_copy(dpart_hbm.at[0], dn0)
  pltpu.sync_copy(dpart_hbm.at[1], dn1)

  @pl.loop(0, NPAD, step=16)
  def _(i):
    dn0[pl.ds(i, 16)] = dn0[pl.ds(i, 16)] + dn1[pl.ds(i, 16)] + 1e-16

  _zero_vmem(dn1, 5120)
  pltpu.sync_copy(dn1.at[pl.ds(0, 5120)],
                  aggp_sp.at[pl.ds(sid * 5120, 5120)])
  plsc.subcore_barrier()

  sems = (sem0, sem1)
  dbufs = (dbuf0, dbuf1)
  sbufs = (sbuf0, sbuf1)
  ebufs = (ebuf0, ebuf1)

  def _issue_b1(t):
    b = jnp.minimum(wid + NW * t, NB - 1)
    base = b * B
    p = t % 2
    hd = pltpu.async_copy(ei_hbm.at[pl.ds(E + base, B)], dbufs[p], sems[p])
    hs = pltpu.async_copy(ei_hbm.at[pl.ds(base, B)], sbufs[p], sems[p])
    he = pltpu.async_copy(es_hbm.at[pl.ds(base, B)], ebufs[p], sems[p])
    return hd, hs, he

  hh = _issue_b1(0)
  for t in range(MAXT):
    b = wid + NW * t
    for h in hh:
      h.wait()
    if t + 1 < MAXT:
      hh = _issue_b1(t + 1)
    dbp = dbufs[t % 2]
    sbp = sbufs[t % 2]
    ebp = ebufs[t % 2]

    @pl.when(b < NB)
    def _():
      @pl.loop(0, B // 16)
      def _(g):
        d16 = dbp[pl.ds(g * 16, 16)]
        s16 = sbp[pl.ds(g * 16, 16)]
        e16 = ebp[pl.ds(g * 16, 16)]
        a16 = e16 / plsc.load_gather(dn0, [d16])
        for c in range(6):
          pc = plsc.load_gather(p6_v, [jnp.full((16,), c, jnp.int32), s16])
          vbuf[pl.ds(g * 96 + c * 16, 16)] = a16 * pc
          ibuf[pl.ds(g * 96 + c * 16, 16)] = d16 * 8 + (c + 2)
      pltpu.sync_copy(vbuf, aggp_sp.at[ibuf], add=True)

  plsc.subcore_barrier()
  pltpu.sync_copy(aggp_sp.at[pl.ds(sid * 5120, 5120)],
                  aggp_hbm.at[cid, pl.ds(sid * 5120, 5120)])


@functools.partial(
    pl.kernel,
    out_type=(
        jax.ShapeDtypeStruct((NC, T * NPAD), jnp.float32),
        jax.ShapeDtypeStruct((T,), jnp.int32),
    ),
    mesh=_mesh,
    scratch_types=[
        pltpu.VMEM((NPAD,), jnp.float32),
        pltpu.VMEM((NPAD,), jnp.float32),
        pltpu.VMEM((N,), jnp.int32),
        pltpu.VMEM((T,), jnp.int32),
        pltpu.VMEM((T,), jnp.int32),
        pltpu.VMEM((B,), jnp.int32),
        pltpu.VMEM((B,), jnp.int32),
        pltpu.VMEM((B,), jnp.int32),
        pltpu.VMEM((B,), jnp.int32),
        pltpu.VMEM((B,), jnp.float32),
        pltpu.VMEM((B,), jnp.float32),
        pltpu.VMEM((B,), jnp.float32),
        pltpu.VMEM((B,), jnp.int32),
        pltpu.VMEM_SHARED((WMAT_PAD,), jnp.float32),
        pltpu.SemaphoreType.DMA,
        pltpu.SemaphoreType.DMA,
    ],
    compiler_params=_sc_params,
)
def _sc_pass_b2(ei_hbm, es_hbm, dpart_hbm, clue_hbm,
                wmat_hbm, slotpos_hbm,
                dn0, dn1, slot_v, cluebuf, spbuf,
                dbuf0, dbuf1, sbuf0, sbuf1, ebuf0, ebuf1,
                abuf, wibuf, wmat_sp, sem0, sem1):
  cid = lax.axis_index("c")
  sid = lax.axis_index("s")
  wid = sid * NC + cid

  pltpu.sync_copy(dpart_hbm.at[0], dn0)
  pltpu.sync_copy(dpart_hbm.at[1], dn1)

  @pl.loop(0, NPAD, step=16)
  def _(i):
    dn0[pl.ds(i, 16)] = dn0[pl.ds(i, 16)] + dn1[pl.ds(i, 16)] + 1e-16

  pltpu.sync_copy(clue_hbm, cluebuf)
  f64 = jnp.full((16,), 64, jnp.int32)

  @pl.loop(0, N, step=16)
  def _(i):
    slot_v[pl.ds(i, 16)] = f64

  lane0 = lax.iota(jnp.int32, 16) == 0
  for tt in range(T):
    node = plsc.load_gather(cluebuf, [jnp.full((16,), tt, jnp.int32)])
    plsc.store_scatter(slot_v, [node], jnp.full((16,), tt, jnp.int32),
                       mask=lane0)

  _zero_vmem(dn1, NPAD)
  for k in range(4):
    pltpu.sync_copy(dn1, wmat_sp.at[pl.ds(sid * 41600 + k * NPAD, NPAD)])
  pltpu.sync_copy(dn1.at[pl.ds(0, 640)],
                  wmat_sp.at[pl.ds(sid * 41600 + 4 * NPAD, 640)])
  plsc.subcore_barrier()

  @pl.when(wid == 0)
  def _():
    for q in range(T // 16):
      c16 = cluebuf[pl.ds(q * 16, 16)]
      spbuf[pl.ds(q * 16, 16)] = plsc.load_gather(slot_v, [c16])
    pltpu.sync_copy(spbuf, slotpos_hbm)

  sems = (sem0, sem1)
  dbufs = (dbuf0, dbuf1)
  sbufs = (sbuf0, sbuf1)
  ebufs = (ebuf0, ebuf1)

  def _issue_b2(t):
    b = jnp.minimum(wid + NW * t, NB - 1)
    base = b * B
    p = t % 2
    hd = pltpu.async_copy(ei_hbm.at[pl.ds(E + base, B)], dbufs[p], sems[p])
    hs = pltpu.async_copy(ei_hbm.at[pl.ds(base, B)], sbufs[p], sems[p])
    he = pltpu.async_copy(es_hbm.at[pl.ds(base, B)], ebufs[p], sems[p])
    return hd, hs, he

  hh = _issue_b2(0)
  for t in range(MAXT):
    b = wid + NW * t
    for h in hh:
      h.wait()
    if t + 1 < MAXT:
      hh = _issue_b2(t + 1)
    dbp = dbufs[t % 2]
    sbp = sbufs[t % 2]
    ebp = ebufs[t % 2]

    @pl.when(b < NB)
    def _():
      @pl.loop(0, B // 16)
      def _(g):
        d16 = dbp[pl.ds(g * 16, 16)]
        s16 = sbp[pl.ds(g * 16, 16)]
        e16 = ebp[pl.ds(g * 16, 16)]
        abuf[pl.ds(g * 16, 16)] = e16 / plsc.load_gather(dn0, [d16])
        st16 = plsc.load_gather(slot_v, [d16])
        wibuf[pl.ds(g * 16, 16)] = st16 * NPAD + s16
      pltpu.sync_copy(abuf, wmat_sp.at[wibuf], add=True)

  plsc.subcore_barrier()
  pltpu.sync_copy(wmat_sp.at[pl.ds(sid * 40960, 40960)],
                  wmat_hbm.at[cid, pl.ds(sid * 40960, 40960)])


def _k1_body(x_ref, wg_ref, bg_ref, wp_ref, wgx_ref, pt_ref, q_ref):
  xb = x_ref[...]
  wgx = lax.dot_general(xb, wg_ref[...], (((1,), (1,)), ((), ())),
                        preferred_element_type=jnp.float32) + bg_ref[...]
  wgx_ref[pl.ds(0, N), :] = wgx
  wgx_ref[pl.ds(N, NPAD - N), :] = jnp.zeros((NPAD - N, D), jnp.float32)
  pt_ref[...] = lax.dot_general(wp_ref[...], wgx, (((1,), (1,)), ((), ())),
                                preferred_element_type=jnp.float32)
  q_ref[...] = lax.dot_general(xb, wp_ref[...], (((1,), (1,)), ((), ())),
                               preferred_element_type=jnp.float32)


def _k1(x, W_g, b_g2, Wproj):
  return pl.pallas_call(
      _k1_body,
      out_shape=[
          jax.ShapeDtypeStruct((NPAD, D), jnp.float32),
          jax.ShapeDtypeStruct((8, N), jnp.float32),
          jax.ShapeDtypeStruct((N, 8), jnp.float32),
      ],
  )(x, W_g, b_g2, Wproj)


def _k3_body(x_ref, wgx_ref, w_ref, p_ref, q_ref,
             clue_ref, sp_ref, wih_ref, whh_ref, bih_ref, bhh_ref,
             wc2_ref, we2_ref, bc_ref, be_ref,
             oc_ref, oe_ref, aggc_ref, clues_ref, gi_ref):
  wmat = w_ref[0] + w_ref[1]
  aggc_ref[...] = lax.dot_general(wmat, wgx_ref[...],
                                  (((1,), (0,)), ((), ())),
                                  preferred_element_type=jnp.float32)

  def build_row(t, _):
    xr = x_ref[pl.ds(clue_ref[t], 1), :]
    cr = aggc_ref[pl.ds(sp_ref[t], 1), :]
    clues_ref[pl.ds(t, 1), :] = xr + cr
    return 0

  lax.fori_loop(0, T, build_row, 0)

  gi_ref[...] = lax.dot_general(clues_ref[...], wih_ref[...],
                                (((1,), (1,)), ((), ())),
                                preferred_element_type=jnp.float32) + bih_ref[...]

  def gru_step(t, h):
    gh = lax.dot_general(h, whh_ref[...], (((1,), (1,)), ((), ())),
                         preferred_element_type=jnp.float32) + bhh_ref[...]
    gi = gi_ref[pl.ds(t, 1), :]
    r = jax.nn.sigmoid(gi[:, :H] + gh[:, :H])
    z = jax.nn.sigmoid(gi[:, H:2 * H] + gh[:, H:2 * H])
    ng = jnp.tanh(gi[:, 2 * H:] + r * gh[:, 2 * H:])
    return (1.0 - z) * ng + z * h

  h = lax.fori_loop(0, T, gru_step, jnp.zeros((1, H), jnp.float32))

  cc = lax.dot_general(h, wc2_ref[...], (((1,), (1,)), ((), ())),
                       preferred_element_type=jnp.float32) + bc_ref[...]
  ec = lax.dot_general(h, we2_ref[...], (((1,), (1,)), ((), ())),
                       preferred_element_type=jnp.float32) + be_ref[...]

  q = q_ref[...]
  p0 = p_ref[0][:N, :]
  p1 = p_ref[1][:N, :]
  oc_ref[...] = q[:, 2:5] + p0[:, 2:5] + p1[:, 2:5] + cc
  oe_ref[...] = q[:, 5:8] + p0[:, 5:8] + p1[:, 5:8] + ec


def _k3(x, wgx, w, p, q, clue_idx, slotpos,
        W_ih, W_hh, b_ih2, b_hh2, wc2, we2, bc2, be2):
  sspec = pl.BlockSpec(memory_space=pltpu.SMEM)
  return pl.pallas_call(
      _k3_body,
      in_specs=[
          pl.BlockSpec(memory_space=pltpu.VMEM),
          pl.BlockSpec(memory_space=pltpu.VMEM),
          pl.BlockSpec(memory_space=pltpu.VMEM),
          pl.BlockSpec(memory_space=pltpu.VMEM),
          pl.BlockSpec(memory_space=pltpu.VMEM),
          sspec,
          sspec,
          pl.BlockSpec(memory_space=pltpu.VMEM),
          pl.BlockSpec(memory_space=pltpu.VMEM),
          pl.BlockSpec(memory_space=pltpu.VMEM),
          pl.BlockSpec(memory_space=pltpu.VMEM),
          pl.BlockSpec(memory_space=pltpu.VMEM),
          pl.BlockSpec(memory_space=pltpu.VMEM),
          pl.BlockSpec(memory_space=pltpu.VMEM),
          pl.BlockSpec(memory_space=pltpu.VMEM),
      ],
      out_specs=[
          pl.BlockSpec(memory_space=pltpu.VMEM),
          pl.BlockSpec(memory_space=pltpu.VMEM),
      ],
      out_shape=[
          jax.ShapeDtypeStruct((N, 3), jnp.float32),
          jax.ShapeDtypeStruct((N, 3), jnp.float32),
      ],
      scratch_shapes=[
          pltpu.VMEM((T, D), jnp.float32),
          pltpu.VMEM((T, D), jnp.float32),
          pltpu.VMEM((T, 3 * H), jnp.float32),
      ],
  )(x, wgx, w, p, q, clue_idx, slotpos,
    W_ih, W_hh, b_ih2, b_hh2, wc2, we2, bc2, be2)


def kernel(x, edge_index, clue_idx, W_g, b_g, alpha_left, alpha_right,
           W_ih, W_hh, b_ih, b_hh, W_cause, b_cause, W_effect, b_effect):
  Wproj = jnp.concatenate(
      [alpha_left[None, :], alpha_right[None, :],
       W_cause[:, :D], W_effect[:, :D]], axis=0)

  wgx, pt, q = _k1(x, W_g, b_g[None, :], Wproj)

  ei = edge_index.reshape(2 * E)
  es, dparts = _sc_pass_a(ei, pt)
  aggp = _sc_pass_b1(ei, es, pt, dparts)
  wmatp, slotpos = _sc_pass_b2(ei, es, dparts, clue_idx)

  w = wmatp.reshape(NC, T, NPAD)
  p = aggp.reshape(NC, NPAD, 8)

  oc, oe = _k3(x, wgx, w, p, q, clue_idx, slotpos,
               W_ih, W_hh, b_ih[None, :], b_hh[None, :],
               W_cause[:, D:], W_effect[:, D:],
               b_cause[None, :], b_effect[None, :])
  return oc, oe


# speedup vs baseline: 19.6247x; 1.0403x over previous
import dataclasses
import functools

import jax
import jax.numpy as jnp
from jax import lax
from jax.experimental import pallas as pl
from jax.experimental.pallas import tpu as pltpu
from jax.experimental.pallas import tpu_sc as plsc

N = 10000
E = 160000
D = 256
H = 128
T = 64

NC = 2
NS = 16
NW = NC * NS
B = 640
NB = E // B
MAXT = (NB + NW - 1) // NW

NPAD = 10240
AGG_PAD = 81920
WMAT_PAD = 665600

_mesh = plsc.VectorSubcoreMesh(core_axis_name="c", subcore_axis_name="s")

_sc_params = pltpu.CompilerParams()
if "needs_layout_passes" in pltpu.CompilerParams.__dataclass_fields__:
  _sc_params = dataclasses.replace(_sc_params, needs_layout_passes=False)


def _zero_vmem(ref, n):
  z = jnp.zeros((16,), ref.dtype)

  @pl.loop(0, n, step=16)
  def _(i):
    ref[pl.ds(i, 16)] = z


@functools.partial(
    pl.kernel,
    out_type=(
        jax.ShapeDtypeStruct((E,), jnp.float32),
        jax.ShapeDtypeStruct((NC, NPAD), jnp.float32),
    ),
    mesh=_mesh,
    scratch_types=[
        pltpu.VMEM((N,), jnp.float32),
        pltpu.VMEM((N,), jnp.float32),
        pltpu.VMEM((B,), jnp.int32),
        pltpu.VMEM((B,), jnp.int32),
        pltpu.VMEM((B,), jnp.int32),
        pltpu.VMEM((B,), jnp.int32),
        pltpu.VMEM((B,), jnp.float32),
        pltpu.VMEM((640,), jnp.float32),
        pltpu.VMEM_SHARED((NPAD,), jnp.float32),
        pltpu.SemaphoreType.DMA,
        pltpu.SemaphoreType.DMA,
    ],
    compiler_params=_sc_params,
)
def _sc_pass_a(ei_hbm, pt_hbm, es_hbm, dpart_hbm,
               sl_v, sr_v, dbuf0, dbuf1, sbuf0, sbuf1, ebuf, zbuf, denom_sp,
               sem0, sem1):
  cid = lax.axis_index("c")
  sid = lax.axis_index("s")
  wid = sid * NC + cid

  pltpu.sync_copy(pt_hbm.at[0], sl_v)
  pltpu.sync_copy(pt_hbm.at[1], sr_v)

  _zero_vmem(zbuf, 640)
  pltpu.sync_copy(zbuf, denom_sp.at[pl.ds(sid * 640, 640)])
  plsc.subcore_barrier()

  sems = (sem0, sem1)
  dbufs = (dbuf0, dbuf1)
  sbufs = (sbuf0, sbuf1)

  def _issue_a(t):
    b = jnp.minimum(wid + NW * t, NB - 1)
    base = b * B
    p = t % 2
    hd = pltpu.async_copy(ei_hbm.at[pl.ds(E + base, B)], dbufs[p], sems[p])
    hs = pltpu.async_copy(ei_hbm.at[pl.ds(base, B)], sbufs[p], sems[p])
    return hd, hs

  hh = _issue_a(0)
  for t in range(MAXT):
    b = wid + NW * t
    for h in hh:
      h.wait()
    if t + 1 < MAXT:
      hh = _issue_a(t + 1)
    dbp = dbufs[t % 2]
    sbp = sbufs[t % 2]

    @pl.when(b < NB)
    def _():
      base = b * B

      @pl.loop(0, B // 16)
      def _(g):
        d16 = dbp[pl.ds(g * 16, 16)]
        s16 = sbp[pl.ds(g * 16, 16)]
        sv = plsc.load_gather(sl_v, [d16]) + plsc.load_gather(sr_v, [s16])
        sv = jnp.where(sv > 0, sv, 0.2 * sv)
        ebuf[pl.ds(g * 16, 16)] = jnp.exp(sv)
      pltpu.sync_copy(ebuf, es_hbm.at[pl.ds(base, B)])
      pltpu.sync_copy(ebuf, denom_sp.at[dbp], add=True)

  plsc.subcore_barrier()
  pltpu.sync_copy(denom_sp.at[pl.ds(sid * 640, 640)],
                  dpart_hbm.at[cid, pl.ds(sid * 640, 640)])


@functools.partial(
    pl.kernel,
    out_type=jax.ShapeDtypeStruct((NC, AGG_PAD), jnp.float32),
    mesh=_mesh,
    scratch_types=[
        pltpu.VMEM((6, N), jnp.float32),
        pltpu.VMEM((NPAD,), jnp.float32),
        pltpu.VMEM((NPAD,), jnp.float32),
        pltpu.VMEM((B,), jnp.int32),
        pltpu.VMEM((B,), jnp.int32),
        pltpu.VMEM((B,), jnp.int32),
        pltpu.VMEM((B,), jnp.int32),
        pltpu.VMEM((B,), jnp.float32),
        pltpu.VMEM((B,), jnp.float32),
        pltpu.VMEM((6 * B,), jnp.float32),
        pltpu.VMEM((6 * B,), jnp.int32),
        pltpu.VMEM_SHARED((AGG_PAD,), jnp.float32),
        pltpu.SemaphoreType.DMA,
        pltpu.SemaphoreType.DMA,
    ],
    compiler_params=_sc_params,
)
def _sc_pass_b1(ei_hbm, es_hbm, pt_hbm, dpart_hbm,
                aggp_hbm,
                p6_v, dn0, dn1, dbuf0, dbuf1, sbuf0, sbuf1, ebuf0, ebuf1,
                vbuf, ibuf, aggp_sp, sem0, sem1):
  cid = lax.axis_index("c")
  sid = lax.axis_index("s")
  wid = sid * NC + cid

  for c in range(6):
    pltpu.sync_copy(pt_hbm.at[2 + c], p6_v.at[c])
  pltpu.sync_copy(dpart_hbm.at[0], dn0)
  pltpu.sync_copy(dpart_hbm.at[1], dn1)

  @pl.loop(0, NPAD, step=16)
  def _(i):
    dn0[pl.ds(i, 16)] = dn0[pl.ds(i, 16)] + dn1[pl.ds(i, 16)] + 1e-16

  _zero_vmem(dn1, 5120)
  pltpu.sync_copy(dn1.at[pl.ds(0, 5120)],
                  aggp_sp.at[pl.ds(sid * 5120, 5120)])
  plsc.subcore_barrier()

  sems = (sem0, sem1)
  dbufs = (dbuf0, dbuf1)
  sbufs = (sbuf0, sbuf1)
  ebufs = (ebuf0, ebuf1)

  def _issue_b1(t):
    b = jnp.minimum(wid + NW * t, NB - 1)
    base = b * B
    p = t % 2
    hd = pltpu.async_copy(ei_hbm.at[pl.ds(E + base, B)], dbufs[p], sems[p])
    hs = pltpu.async_copy(ei_hbm.at[pl.ds(base, B)], sbufs[p], sems[p])
    he = pltpu.async_copy(es_hbm.at[pl.ds(base, B)], ebufs[p], sems[p])
    return hd, hs, he

  hh = _issue_b1(0)
  for t in range(MAXT):
    b = wid + NW * t
    for h in hh:
      h.wait()
    if t + 1 < MAXT:
      hh = _issue_b1(t + 1)
    dbp = dbufs[t % 2]
    sbp = sbufs[t % 2]
    ebp = ebufs[t % 2]

    @pl.when(b < NB)
    def _():
      @pl.loop(0, B // 16)
      def _(g):
        d16 = dbp[pl.ds(g * 16, 16)]
        s16 = sbp[pl.ds(g * 16, 16)]
        e16 = ebp[pl.ds(g * 16, 16)]
        a16 = e16 / plsc.load_gather(dn0, [d16])
        for c in range(6):
          pc = plsc.load_gather(p6_v, [jnp.full((16,), c, jnp.int32), s16])
          vbuf[pl.ds(g * 96 + c * 16, 16)] = a16 * pc
          ibuf[pl.ds(g * 96 + c * 16, 16)] = d16 * 8 + (c + 2)
      pltpu.sync_copy(vbuf, aggp_sp.at[ibuf], add=True)

  plsc.subcore_barrier()
  pltpu.sync_copy(aggp_sp.at[pl.ds(sid * 5120, 5120)],
                  aggp_hbm.at[cid, pl.ds(sid * 5120, 5120)])


@functools.partial(
    pl.kernel,
    out_type=(
        jax.ShapeDtypeStruct((T, NC, NPAD), jnp.float32),
        jax.ShapeDtypeStruct((T,), jnp.int32),
    ),
    mesh=_mesh,
    scratch_types=[
        pltpu.VMEM((NPAD,), jnp.float32),
        pltpu.VMEM((NPAD,), jnp.float32),
        pltpu.VMEM((N,), jnp.int32),
        pltpu.VMEM((T,), jnp.int32),
        pltpu.VMEM((T,), jnp.int32),
        pltpu.VMEM((B,), jnp.int32),
        pltpu.VMEM((B,), jnp.int32),
        pltpu.VMEM((B,), jnp.int32),
        pltpu.VMEM((B,), jnp.int32),
        pltpu.VMEM((B,), jnp.float32),
        pltpu.VMEM((B,), jnp.float32),
        pltpu.VMEM((B,), jnp.float32),
        pltpu.VMEM((B,), jnp.int32),
        pltpu.VMEM_SHARED((WMAT_PAD,), jnp.float32),
        pltpu.SemaphoreType.DMA,
        pltpu.SemaphoreType.DMA,
    ],
    compiler_params=_sc_params,
)
def _sc_pass_b2(ei_hbm, es_hbm, dpart_hbm, clue_hbm,
                wmat_hbm, slotpos_hbm,
                dn0, dn1, slot_v, cluebuf, spbuf,
                dbuf0, dbuf1, sbuf0, sbuf1, ebuf0, ebuf1,
                abuf, wibuf, wmat_sp, sem0, sem1):
  cid = lax.axis_index("c")
  sid = lax.axis_index("s")
  wid = sid * NC + cid

  pltpu.sync_copy(dpart_hbm.at[0], dn0)
  pltpu.sync_copy(dpart_hbm.at[1], dn1)

  @pl.loop(0, NPAD, step=16)
  def _(i):
    dn0[pl.ds(i, 16)] = dn0[pl.ds(i, 16)] + dn1[pl.ds(i, 16)] + 1e-16

  pltpu.sync_copy(clue_hbm, cluebuf)
  f64 = jnp.full((16,), 64, jnp.int32)

  @pl.loop(0, N, step=16)
  def _(i):
    slot_v[pl.ds(i, 16)] = f64

  lane0 = lax.iota(jnp.int32, 16) == 0
  for tt in range(T):
    node = plsc.load_gather(cluebuf, [jnp.full((16,), tt, jnp.int32)])
    plsc.store_scatter(slot_v, [node], jnp.full((16,), tt, jnp.int32),
                       mask=lane0)

  _zero_vmem(dn1, NPAD)
  for k in range(4):
    pltpu.sync_copy(dn1, wmat_sp.at[pl.ds(sid * 41600 + k * NPAD, NPAD)])
  pltpu.sync_copy(dn1.at[pl.ds(0, 640)],
                  wmat_sp.at[pl.ds(sid * 41600 + 4 * NPAD, 640)])
  plsc.subcore_barrier()

  @pl.when(wid == 0)
  def _():
    for q in range(T // 16):
      c16 = cluebuf[pl.ds(q * 16, 16)]
      spbuf[pl.ds(q * 16, 16)] = plsc.load_gather(slot_v, [c16])
    pltpu.sync_copy(spbuf, slotpos_hbm)

  sems = (sem0, sem1)
  dbufs = (dbuf0, dbuf1)
  sbufs = (sbuf0, sbuf1)
  ebufs = (ebuf0, ebuf1)

  def _issue_b2(t):
    b = jnp.minimum(wid + NW * t, NB - 1)
    base = b * B
    p = t % 2
    hd = pltpu.async_copy(ei_hbm.at[pl.ds(E + base, B)], dbufs[p], sems[p])
    hs = pltpu.async_copy(ei_hbm.at[pl.ds(base, B)], sbufs[p], sems[p])
    he = pltpu.async_copy(es_hbm.at[pl.ds(base, B)], ebufs[p], sems[p])
    return hd, hs, he

  hh = _issue_b2(0)
  for t in range(MAXT):
    b = wid + NW * t
    for h in hh:
      h.wait()
    if t + 1 < MAXT:
      hh = _issue_b2(t + 1)
    dbp = dbufs[t % 2]
    sbp = sbufs[t % 2]
    ebp = ebufs[t % 2]

    @pl.when(b < NB)
    def _():
      @pl.loop(0, B // 16)
      def _(g):
        d16 = dbp[pl.ds(g * 16, 16)]
        s16 = sbp[pl.ds(g * 16, 16)]
        e16 = ebp[pl.ds(g * 16, 16)]
        abuf[pl.ds(g * 16, 16)] = e16 / plsc.load_gather(dn0, [d16])
        st16 = plsc.load_gather(slot_v, [d16])
        wibuf[pl.ds(g * 16, 16)] = st16 * NPAD + s16
      pltpu.sync_copy(abuf, wmat_sp.at[wibuf], add=True)

  plsc.subcore_barrier()
  for r in range(4):
    pltpu.sync_copy(wmat_sp.at[pl.ds((sid * 4 + r) * NPAD, NPAD)],
                    wmat_hbm.at[sid * 4 + r, cid])


def _k1_body(x_ref, wg_ref, bg_ref, wp_ref, wgx_ref, pt_ref, q_ref):
  xb = x_ref[...]
  wgx = lax.dot_general(xb, wg_ref[...], (((1,), (1,)), ((), ())),
                        preferred_element_type=jnp.float32) + bg_ref[...]
  wgx_ref[pl.ds(0, N), :] = wgx
  wgx_ref[pl.ds(N, NPAD - N), :] = jnp.zeros((NPAD - N, D), jnp.float32)
  pt_ref[...] = lax.dot_general(wp_ref[...], wgx, (((1,), (1,)), ((), ())),
                                preferred_element_type=jnp.float32)
  q_ref[...] = lax.dot_general(xb, wp_ref[...], (((1,), (1,)), ((), ())),
                               preferred_element_type=jnp.float32)


def _k1(x, W_g, b_g2, Wproj):
  return pl.pallas_call(
      _k1_body,
      out_shape=[
          jax.ShapeDtypeStruct((NPAD, D), jnp.float32),
          jax.ShapeDtypeStruct((8, N), jnp.float32),
          jax.ShapeDtypeStruct((N, 8), jnp.float32),
      ],
  )(x, W_g, b_g2, Wproj)


def _k3_body(x_ref, wgx_ref, w_ref, p_ref, q_ref,
             clue_ref, sp_ref, wih_ref, whh_ref, bih_ref, bhh_ref,
             wc2_ref, we2_ref, bc_ref, be_ref,
             oc_ref, oe_ref, aggc_ref, clues_ref, gi_ref):
  wv = w_ref[...]
  wmat = wv[:, 0, :] + wv[:, 1, :]
  aggc_ref[...] = lax.dot_general(wmat, wgx_ref[...],
                                  (((1,), (0,)), ((), ())),
                                  preferred_element_type=jnp.float32)

  def build_row(t, _):
    xr = x_ref[pl.ds(clue_ref[t], 1), :]
    cr = aggc_ref[pl.ds(sp_ref[t], 1), :]
    clues_ref[pl.ds(t, 1), :] = xr + cr
    return 0

  lax.fori_loop(0, T, build_row, 0)

  gi_ref[...] = lax.dot_general(clues_ref[...], wih_ref[...],
                                (((1,), (1,)), ((), ())),
                                preferred_element_type=jnp.float32) + bih_ref[...]

  def gru_step(t, h):
    gh = lax.dot_general(h, whh_ref[...], (((1,), (1,)), ((), ())),
                         preferred_element_type=jnp.float32) + bhh_ref[...]
    gi = gi_ref[pl.ds(t, 1), :]
    r = jax.nn.sigmoid(gi[:, :H] + gh[:, :H])
    z = jax.nn.sigmoid(gi[:, H:2 * H] + gh[:, H:2 * H])
    ng = jnp.tanh(gi[:, 2 * H:] + r * gh[:, 2 * H:])
    return (1.0 - z) * ng + z * h

  h = lax.fori_loop(0, T, gru_step, jnp.zeros((1, H), jnp.float32))

  cc = lax.dot_general(h, wc2_ref[...], (((1,), (1,)), ((), ())),
                       preferred_element_type=jnp.float32) + bc_ref[...]
  ec = lax.dot_general(h, we2_ref[...], (((1,), (1,)), ((), ())),
                       preferred_element_type=jnp.float32) + be_ref[...]

  q = q_ref[...]
  p0 = p_ref[0][:N, :]
  p1 = p_ref[1][:N, :]
  oc_ref[...] = q[:, 2:5] + p0[:, 2:5] + p1[:, 2:5] + cc
  oe_ref[...] = q[:, 5:8] + p0[:, 5:8] + p1[:, 5:8] + ec


def _k3(x, wgx, w, p, q, clue_idx, slotpos,
        W_ih, W_hh, b_ih2, b_hh2, wc2, we2, bc2, be2):
  sspec = pl.BlockSpec(memory_space=pltpu.SMEM)
  return pl.pallas_call(
      _k3_body,
      in_specs=[
          pl.BlockSpec(memory_space=pltpu.VMEM),
          pl.BlockSpec(memory_space=pltpu.VMEM),
          pl.BlockSpec(memory_space=pltpu.VMEM),
          pl.BlockSpec(memory_space=pltpu.VMEM),
          pl.BlockSpec(memory_space=pltpu.VMEM),
          sspec,
          sspec,
          pl.BlockSpec(memory_space=pltpu.VMEM),
          pl.BlockSpec(memory_space=pltpu.VMEM),
          pl.BlockSpec(memory_space=pltpu.VMEM),
          pl.BlockSpec(memory_space=pltpu.VMEM),
          pl.BlockSpec(memory_space=pltpu.VMEM),
          pl.BlockSpec(memory_space=pltpu.VMEM),
          pl.BlockSpec(memory_space=pltpu.VMEM),
          pl.BlockSpec(memory_space=pltpu.VMEM),
      ],
      out_specs=[
          pl.BlockSpec(memory_space=pltpu.VMEM),
          pl.BlockSpec(memory_space=pltpu.VMEM),
      ],
      out_shape=[
          jax.ShapeDtypeStruct((N, 3), jnp.float32),
          jax.ShapeDtypeStruct((N, 3), jnp.float32),
      ],
      scratch_shapes=[
          pltpu.VMEM((T, D), jnp.float32),
          pltpu.VMEM((T, D), jnp.float32),
          pltpu.VMEM((T, 3 * H), jnp.float32),
      ],
  )(x, wgx, w, p, q, clue_idx, slotpos,
    W_ih, W_hh, b_ih2, b_hh2, wc2, we2, bc2, be2)


def kernel(x, edge_index, clue_idx, W_g, b_g, alpha_left, alpha_right,
           W_ih, W_hh, b_ih, b_hh, W_cause, b_cause, W_effect, b_effect):
  Wproj = jnp.concatenate(
      [alpha_left[None, :], alpha_right[None, :],
       W_cause[:, :D], W_effect[:, :D]], axis=0)

  wgx, pt, q = _k1(x, W_g, b_g[None, :], Wproj)

  ei = edge_index.reshape(2 * E)
  es, dparts = _sc_pass_a(ei, pt)
  aggp = _sc_pass_b1(ei, es, pt, dparts)
  wmatp, slotpos = _sc_pass_b2(ei, es, dparts, clue_idx)

  w = wmatp
  p = aggp.reshape(NC, NPAD, 8)

  oc, oe = _k3(x, wgx, w, p, q, clue_idx, slotpos,
               W_ih, W_hh, b_ih[None, :], b_hh[None, :],
               W_cause[:, D:], W_effect[:, D:],
               b_cause[None, :], b_effect[None, :])
  return oc, oe


# speedup vs baseline: 19.7411x; 1.0059x over previous
import dataclasses
import functools

import jax
import jax.numpy as jnp
from jax import lax
from jax.experimental import pallas as pl
from jax.experimental.pallas import tpu as pltpu
from jax.experimental.pallas import tpu_sc as plsc

N = 10000
E = 160000
D = 256
H = 128
T = 64

NC = 2
NS = 16
NW = NC * NS
B = 640
NB = E // B
MAXT = (NB + NW - 1) // NW

NPAD = 10240
AGG_PAD = 81920
WMAT_PAD = 665600

_mesh = plsc.VectorSubcoreMesh(core_axis_name="c", subcore_axis_name="s")

_sc_params = pltpu.CompilerParams()
if "needs_layout_passes" in pltpu.CompilerParams.__dataclass_fields__:
  _sc_params = dataclasses.replace(_sc_params, needs_layout_passes=False)


def _zero_vmem(ref, n):
  z = jnp.zeros((16,), ref.dtype)

  @pl.loop(0, n, step=16)
  def _(i):
    ref[pl.ds(i, 16)] = z


@functools.partial(
    pl.kernel,
    out_type=(
        jax.ShapeDtypeStruct((E,), jnp.float32),
        jax.ShapeDtypeStruct((NC, NPAD), jnp.float32),
    ),
    mesh=_mesh,
    scratch_types=[
        pltpu.VMEM((N,), jnp.float32),
        pltpu.VMEM((N,), jnp.float32),
        pltpu.VMEM((B,), jnp.int32),
        pltpu.VMEM((B,), jnp.int32),
        pltpu.VMEM((B,), jnp.int32),
        pltpu.VMEM((B,), jnp.int32),
        pltpu.VMEM((B,), jnp.float32),
        pltpu.VMEM((640,), jnp.float32),
        pltpu.VMEM_SHARED((NPAD,), jnp.float32),
        pltpu.SemaphoreType.DMA,
        pltpu.SemaphoreType.DMA,
    ],
    compiler_params=_sc_params,
)
def _sc_pass_a(ei_hbm, pt_hbm, es_hbm, dpart_hbm,
               sl_v, sr_v, dbuf0, dbuf1, sbuf0, sbuf1, ebuf, zbuf, denom_sp,
               sem0, sem1):
  cid = lax.axis_index("c")
  sid = lax.axis_index("s")
  wid = sid * NC + cid

  pltpu.sync_copy(pt_hbm.at[0], sl_v)
  pltpu.sync_copy(pt_hbm.at[1], sr_v)

  _zero_vmem(zbuf, 640)
  pltpu.sync_copy(zbuf, denom_sp.at[pl.ds(sid * 640, 640)])
  plsc.subcore_barrier()

  sems = (sem0, sem1)
  dbufs = (dbuf0, dbuf1)
  sbufs = (sbuf0, sbuf1)

  def _issue_a(t):
    b = jnp.minimum(wid + NW * t, NB - 1)
    base = b * B
    p = t % 2
    hd = pltpu.async_copy(ei_hbm.at[pl.ds(E + base, B)], dbufs[p], sems[p])
    hs = pltpu.async_copy(ei_hbm.at[pl.ds(base, B)], sbufs[p], sems[p])
    return hd, hs

  hh = _issue_a(0)
  for t in range(MAXT):
    b = wid + NW * t
    for h in hh:
      h.wait()
    if t + 1 < MAXT:
      hh = _issue_a(t + 1)
    dbp = dbufs[t % 2]
    sbp = sbufs[t % 2]

    @pl.when(b < NB)
    def _():
      base = b * B

      @pl.loop(0, B // 16)
      def _(g):
        d16 = dbp[pl.ds(g * 16, 16)]
        s16 = sbp[pl.ds(g * 16, 16)]
        sv = plsc.load_gather(sl_v, [d16]) + plsc.load_gather(sr_v, [s16])
        sv = jnp.where(sv > 0, sv, 0.2 * sv)
        ebuf[pl.ds(g * 16, 16)] = jnp.exp(sv)
      pltpu.sync_copy(ebuf, es_hbm.at[pl.ds(base, B)])
      pltpu.sync_copy(ebuf, denom_sp.at[dbp], add=True)

  plsc.subcore_barrier()
  pltpu.sync_copy(denom_sp.at[pl.ds(sid * 640, 640)],
                  dpart_hbm.at[cid, pl.ds(sid * 640, 640)])


@functools.partial(
    pl.kernel,
    out_type=jax.ShapeDtypeStruct((8, NC, NPAD), jnp.float32),
    mesh=_mesh,
    scratch_types=[
        pltpu.VMEM((6, N), jnp.float32),
        pltpu.VMEM((NPAD,), jnp.float32),
        pltpu.VMEM((NPAD,), jnp.float32),
        pltpu.VMEM((B,), jnp.int32),
        pltpu.VMEM((B,), jnp.int32),
        pltpu.VMEM((B,), jnp.int32),
        pltpu.VMEM((B,), jnp.int32),
        pltpu.VMEM((B,), jnp.float32),
        pltpu.VMEM((B,), jnp.float32),
        pltpu.VMEM((6 * B,), jnp.float32),
        pltpu.VMEM((6 * B,), jnp.int32),
        pltpu.VMEM_SHARED((AGG_PAD,), jnp.float32),
        pltpu.SemaphoreType.DMA,
        pltpu.SemaphoreType.DMA,
    ],
    compiler_params=_sc_params,
)
def _sc_pass_b1(ei_hbm, es_hbm, pt_hbm, dpart_hbm,
                aggp_hbm,
                p6_v, dn0, dn1, dbuf0, dbuf1, sbuf0, sbuf1, ebuf0, ebuf1,
                vbuf, ibuf, aggp_sp, sem0, sem1):
  cid = lax.axis_index("c")
  sid = lax.axis_index("s")
  wid = sid * NC + cid

  for c in range(6):
    pltpu.sync_copy(pt_hbm.at[2 + c], p6_v.at[c])
  pltpu.sync_copy(dpart_hbm.at[0], dn0)
  pltpu.sync_copy(dpart_hbm.at[1], dn1)

  @pl.loop(0, NPAD, step=16)
  def _(i):
    dn0[pl.ds(i, 16)] = dn0[pl.ds(i, 16)] + dn1[pl.ds(i, 16)] + 1e-16

  _zero_vmem(dn1, 5120)
  pltpu.sync_copy(dn1.at[pl.ds(0, 5120)],
                  aggp_sp.at[pl.ds(sid * 5120, 5120)])
  plsc.subcore_barrier()

  sems = (sem0, sem1)
  dbufs = (dbuf0, dbuf1)
  sbufs = (sbuf0, sbuf1)
  ebufs = (ebuf0, ebuf1)

  def _issue_b1(t):
    b = jnp.minimum(wid + NW * t, NB - 1)
    base = b * B
    p = t % 2
    hd = pltpu.async_copy(ei_hbm.at[pl.ds(E + base, B)], dbufs[p], sems[p])
    hs = pltpu.async_copy(ei_hbm.at[pl.ds(base, B)], sbufs[p], sems[p])
    he = pltpu.async_copy(es_hbm.at[pl.ds(base, B)], ebufs[p], sems[p])
    return hd, hs, he

  hh = _issue_b1(0)
  for t in range(MAXT):
    b = wid + NW * t
    for h in hh:
      h.wait()
    if t + 1 < MAXT:
      hh = _issue_b1(t + 1)
    dbp = dbufs[t % 2]
    sbp = sbufs[t % 2]
    ebp = ebufs[t % 2]

    @pl.when(b < NB)
    def _():
      @pl.loop(0, B // 16)
      def _(g):
        d16 = dbp[pl.ds(g * 16, 16)]
        s16 = sbp[pl.ds(g * 16, 16)]
        e16 = ebp[pl.ds(g * 16, 16)]
        a16 = e16 / plsc.load_gather(dn0, [d16])
        for c in range(6):
          pc = plsc.load_gather(p6_v, [jnp.full((16,), c, jnp.int32), s16])
          vbuf[pl.ds(g * 96 + c * 16, 16)] = a16 * pc
          ibuf[pl.ds(g * 96 + c * 16, 16)] = (c + 2) * NPAD + d16
      pltpu.sync_copy(vbuf, aggp_sp.at[ibuf], add=True)

  plsc.subcore_barrier()
  pltpu.sync_copy(aggp_sp.at[pl.ds(sid * 5120, 5120)],
                  aggp_hbm.at[sid // 2, cid, pl.ds((sid % 2) * 5120, 5120)])


@functools.partial(
    pl.kernel,
    out_type=(
        jax.ShapeDtypeStruct((T, NC, NPAD), jnp.float32),
        jax.ShapeDtypeStruct((T,), jnp.int32),
    ),
    mesh=_mesh,
    scratch_types=[
        pltpu.VMEM((NPAD,), jnp.float32),
        pltpu.VMEM((NPAD,), jnp.float32),
        pltpu.VMEM((N,), jnp.int32),
        pltpu.VMEM((T,), jnp.int32),
        pltpu.VMEM((T,), jnp.int32),
        pltpu.VMEM((B,), jnp.int32),
        pltpu.VMEM((B,), jnp.int32),
        pltpu.VMEM((B,), jnp.int32),
        pltpu.VMEM((B,), jnp.int32),
        pltpu.VMEM((B,), jnp.float32),
        pltpu.VMEM((B,), jnp.float32),
        pltpu.VMEM((B,), jnp.float32),
        pltpu.VMEM((B,), jnp.int32),
        pltpu.VMEM_SHARED((WMAT_PAD,), jnp.float32),
        pltpu.SemaphoreType.DMA,
        pltpu.SemaphoreType.DMA,
    ],
    compiler_params=_sc_params,
)
def _sc_pass_b2(ei_hbm, es_hbm, dpart_hbm, clue_hbm,
                wmat_hbm, slotpos_hbm,
                dn0, dn1, slot_v, cluebuf, spbuf,
                dbuf0, dbuf1, sbuf0, sbuf1, ebuf0, ebuf1,
                abuf, wibuf, wmat_sp, sem0, sem1):
  cid = lax.axis_index("c")
  sid = lax.axis_index("s")
  wid = sid * NC + cid

  pltpu.sync_copy(dpart_hbm.at[0], dn0)
  pltpu.sync_copy(dpart_hbm.at[1], dn1)

  @pl.loop(0, NPAD, step=16)
  def _(i):
    dn0[pl.ds(i, 16)] = dn0[pl.ds(i, 16)] + dn1[pl.ds(i, 16)] + 1e-16

  pltpu.sync_copy(clue_hbm, cluebuf)
  f64 = jnp.full((16,), 64, jnp.int32)

  @pl.loop(0, N, step=16)
  def _(i):
    slot_v[pl.ds(i, 16)] = f64

  lane0 = lax.iota(jnp.int32, 16) == 0
  for tt in range(T):
    node = plsc.load_gather(cluebuf, [jnp.full((16,), tt, jnp.int32)])
    plsc.store_scatter(slot_v, [node], jnp.full((16,), tt, jnp.int32),
                       mask=lane0)

  _zero_vmem(dn1, NPAD)
  for k in range(4):
    pltpu.sync_copy(dn1, wmat_sp.at[pl.ds(sid * 41600 + k * NPAD, NPAD)])
  pltpu.sync_copy(dn1.at[pl.ds(0, 640)],
                  wmat_sp.at[pl.ds(sid * 41600 + 4 * NPAD, 640)])
  plsc.subcore_barrier()

  @pl.when(wid == 0)
  def _():
    for q in range(T // 16):
      c16 = cluebuf[pl.ds(q * 16, 16)]
      spbuf[pl.ds(q * 16, 16)] = plsc.load_gather(slot_v, [c16])
    pltpu.sync_copy(spbuf, slotpos_hbm)

  sems = (sem0, sem1)
  dbufs = (dbuf0, dbuf1)
  sbufs = (sbuf0, sbuf1)
  ebufs = (ebuf0, ebuf1)

  def _issue_b2(t):
    b = jnp.minimum(wid + NW * t, NB - 1)
    base = b * B
    p = t % 2
    hd = pltpu.async_copy(ei_hbm.at[pl.ds(E + base, B)], dbufs[p], sems[p])
    hs = pltpu.async_copy(ei_hbm.at[pl.ds(base, B)], sbufs[p], sems[p])
    he = pltpu.async_copy(es_hbm.at[pl.ds(base, B)], ebufs[p], sems[p])
    return hd, hs, he

  hh = _issue_b2(0)
  for t in range(MAXT):
    b = wid + NW * t
    for h in hh:
      h.wait()
    if t + 1 < MAXT:
      hh = _issue_b2(t + 1)
    dbp = dbufs[t % 2]
    sbp = sbufs[t % 2]
    ebp = ebufs[t % 2]

    @pl.when(b < NB)
    def _():
      @pl.loop(0, B // 16)
      def _(g):
        d16 = dbp[pl.ds(g * 16, 16)]
        s16 = sbp[pl.ds(g * 16, 16)]
        e16 = ebp[pl.ds(g * 16, 16)]
        abuf[pl.ds(g * 16, 16)] = e16 / plsc.load_gather(dn0, [d16])
        st16 = plsc.load_gather(slot_v, [d16])
        wibuf[pl.ds(g * 16, 16)] = st16 * NPAD + s16
      pltpu.sync_copy(abuf, wmat_sp.at[wibuf], add=True)

  plsc.subcore_barrier()
  for r in range(4):
    pltpu.sync_copy(wmat_sp.at[pl.ds((sid * 4 + r) * NPAD, NPAD)],
                    wmat_hbm.at[sid * 4 + r, cid])


def _k1_body(x_ref, wg_ref, bg_ref, wp_ref, wgx_ref, pt_ref, q_ref):
  xb = x_ref[...]
  wgx = lax.dot_general(xb, wg_ref[...], (((1,), (1,)), ((), ())),
                        preferred_element_type=jnp.float32) + bg_ref[...]
  wgx_ref[pl.ds(0, N), :] = wgx
  wgx_ref[pl.ds(N, NPAD - N), :] = jnp.zeros((NPAD - N, D), jnp.float32)
  pt_ref[...] = lax.dot_general(wp_ref[...], wgx, (((1,), (1,)), ((), ())),
                                preferred_element_type=jnp.float32)
  q_ref[...] = lax.dot_general(xb, wp_ref[...], (((1,), (1,)), ((), ())),
                               preferred_element_type=jnp.float32)


def _k1(x, W_g, b_g2, Wproj):
  return pl.pallas_call(
      _k1_body,
      out_shape=[
          jax.ShapeDtypeStruct((NPAD, D), jnp.float32),
          jax.ShapeDtypeStruct((8, N), jnp.float32),
          jax.ShapeDtypeStruct((N, 8), jnp.float32),
      ],
  )(x, W_g, b_g2, Wproj)


def _k3_body(x_ref, wgx_ref, w_ref, p_ref, q_ref,
             clue_ref, sp_ref, wih_ref, whh_ref, bih_ref, bhh_ref,
             wc2_ref, we2_ref, bc_ref, be_ref,
             oc_ref, oe_ref, aggc_ref, clues_ref, gi_ref):
  wv = w_ref[...]
  wmat = wv[:, 0, :] + wv[:, 1, :]
  aggc_ref[...] = lax.dot_general(wmat, wgx_ref[...],
                                  (((1,), (0,)), ((), ())),
                                  preferred_element_type=jnp.float32)

  def build_row(t, _):
    xr = x_ref[pl.ds(clue_ref[t], 1), :]
    cr = aggc_ref[pl.ds(sp_ref[t], 1), :]
    clues_ref[pl.ds(t, 1), :] = xr + cr
    return 0

  lax.fori_loop(0, T, build_row, 0)

  gi_ref[...] = lax.dot_general(clues_ref[...], wih_ref[...],
                                (((1,), (1,)), ((), ())),
                                preferred_element_type=jnp.float32) + bih_ref[...]

  def gru_step(t, h):
    gh = lax.dot_general(h, whh_ref[...], (((1,), (1,)), ((), ())),
                         preferred_element_type=jnp.float32) + bhh_ref[...]
    gi = gi_ref[pl.ds(t, 1), :]
    r = jax.nn.sigmoid(gi[:, :H] + gh[:, :H])
    z = jax.nn.sigmoid(gi[:, H:2 * H] + gh[:, H:2 * H])
    ng = jnp.tanh(gi[:, 2 * H:] + r * gh[:, 2 * H:])
    return (1.0 - z) * ng + z * h

  h = lax.fori_loop(0, T, gru_step, jnp.zeros((1, H), jnp.float32))

  cc = lax.dot_general(h, wc2_ref[...], (((1,), (1,)), ((), ())),
                       preferred_element_type=jnp.float32) + bc_ref[...]
  ec = lax.dot_general(h, we2_ref[...], (((1,), (1,)), ((), ())),
                       preferred_element_type=jnp.float32) + be_ref[...]

  q = q_ref[...]
  pv = p_ref[...]
  ccols = pv[2:5, 0, :] + pv[2:5, 1, :]
  ecols = pv[5:8, 0, :] + pv[5:8, 1, :]
  oc_ref[...] = q[:, 2:5] + jnp.transpose(ccols)[:N, :] + cc
  oe_ref[...] = q[:, 5:8] + jnp.transpose(ecols)[:N, :] + ec


def _k3(x, wgx, w, p, q, clue_idx, slotpos,
        W_ih, W_hh, b_ih2, b_hh2, wc2, we2, bc2, be2):
  sspec = pl.BlockSpec(memory_space=pltpu.SMEM)
  return pl.pallas_call(
      _k3_body,
      in_specs=[
          pl.BlockSpec(memory_space=pltpu.VMEM),
          pl.BlockSpec(memory_space=pltpu.VMEM),
          pl.BlockSpec(memory_space=pltpu.VMEM),
          pl.BlockSpec(memory_space=pltpu.VMEM),
          pl.BlockSpec(memory_space=pltpu.VMEM),
          sspec,
          sspec,
          pl.BlockSpec(memory_space=pltpu.VMEM),
          pl.BlockSpec(memory_space=pltpu.VMEM),
          pl.BlockSpec(memory_space=pltpu.VMEM),
          pl.BlockSpec(memory_space=pltpu.VMEM),
          pl.BlockSpec(memory_space=pltpu.VMEM),
          pl.BlockSpec(memory_space=pltpu.VMEM),
          pl.BlockSpec(memory_space=pltpu.VMEM),
          pl.BlockSpec(memory_space=pltpu.VMEM),
      ],
      out_specs=[
          pl.BlockSpec(memory_space=pltpu.VMEM),
          pl.BlockSpec(memory_space=pltpu.VMEM),
      ],
      out_shape=[
          jax.ShapeDtypeStruct((N, 3), jnp.float32),
          jax.ShapeDtypeStruct((N, 3), jnp.float32),
      ],
      scratch_shapes=[
          pltpu.VMEM((T, D), jnp.float32),
          pltpu.VMEM((T, D), jnp.float32),
          pltpu.VMEM((T, 3 * H), jnp.float32),
      ],
  )(x, wgx, w, p, q, clue_idx, slotpos,
    W_ih, W_hh, b_ih2, b_hh2, wc2, we2, bc2, be2)


def kernel(x, edge_index, clue_idx, W_g, b_g, alpha_left, alpha_right,
           W_ih, W_hh, b_ih, b_hh, W_cause, b_cause, W_effect, b_effect):
  Wproj = jnp.concatenate(
      [alpha_left[None, :], alpha_right[None, :],
       W_cause[:, :D], W_effect[:, :D]], axis=0)

  wgx, pt, q = _k1(x, W_g, b_g[None, :], Wproj)

  ei = edge_index.reshape(2 * E)
  es, dparts = _sc_pass_a(ei, pt)
  aggp = _sc_pass_b1(ei, es, pt, dparts)
  wmatp, slotpos = _sc_pass_b2(ei, es, dparts, clue_idx)

  w = wmatp
  p = aggp

  oc, oe = _k3(x, wgx, w, p, q, clue_idx, slotpos,
               W_ih, W_hh, b_ih[None, :], b_hh[None, :],
               W_cause[:, D:], W_effect[:, D:],
               b_cause[None, :], b_effect[None, :])
  return oc, oe
